# baseline (speedup 1.0000x reference)
import functools

import jax
import jax.numpy as jnp
from jax import lax
from jax.experimental import pallas as pl
from jax.experimental.pallas import tpu as pltpu
from jax.experimental.pallas import tpu_sc as plsc

N = 10000
E = 320000
D = 128

NC = 2
NS = 16
NW = NC * NS
EPW = E // NW
GB = 200
GBP = 208
SB = 200
NPAD = 10240
RPT = NPAD // NS

_mesh = plsc.VectorSubcoreMesh(core_axis_name="c", subcore_axis_name="s")


def _pre_kernel(x_ref, pos_ref, wxi_ref, wxj_ref, wdir_ref, b0_ref,
                xa_ref, xb_ref):
    pd = jnp.dot(pos_ref[...], wdir_ref[...], preferred_element_type=jnp.float32)
    xv = x_ref[...]
    xa_ref[...] = (jnp.dot(xv, wxi_ref[...], preferred_element_type=jnp.float32)
                   + pd + b0_ref[...])
    xb_ref[...] = (jnp.dot(xv, wxj_ref[...], preferred_element_type=jnp.float32)
                   - pd)


def _pre(x, pos8, wxi, wxj, wdir8, b0):
    bn = 2000
    grid = N // bn
    return pl.pallas_call(
        _pre_kernel,
        grid=(grid,),
        in_specs=[
            pl.BlockSpec((bn, D), lambda i: (i, 0)),
            pl.BlockSpec((bn, 8), lambda i: (i, 0)),
            pl.BlockSpec((D, D), lambda i: (0, 0)),
            pl.BlockSpec((D, D), lambda i: (0, 0)),
            pl.BlockSpec((8, D), lambda i: (0, 0)),
            pl.BlockSpec((1, D), lambda i: (0, 0)),
        ],
        out_specs=[
            pl.BlockSpec((bn, D), lambda i: (i, 0)),
            pl.BlockSpec((bn, D), lambda i: (i, 0)),
        ],
        out_shape=[
            jax.ShapeDtypeStruct((N, D), jnp.float32),
            jax.ShapeDtypeStruct((N, D), jnp.float32),
        ],
    )(x, pos8, wxi, wxj, wdir8, b0)


def _gather_body(xa_h, xb_h, gi_h, gj_h, px_h, py_h, pz_h, w3_h,
                 h0_h,
                 ia_v, ja_v, ra_v, rb_v, nrm_v, px_v, py_v, pz_v, w3_v,
                 sem_a, sem_b):
    wid = lax.axis_index("s") * NC + lax.axis_index("c")
    pltpu.sync_copy(px_h, px_v)
    pltpu.sync_copy(py_h, py_v)
    pltpu.sync_copy(pz_h, pz_v)
    pltpu.sync_copy(w3_h, w3_v)
    w3r = [w3_v[pl.ds(s * 16, 16)] for s in range(D // 16)]
    ia_v[pl.ds(GB - GB % 16, 16)] = jnp.zeros((16,), jnp.int32)
    ja_v[pl.ds(GB - GB % 16, 16)] = jnp.zeros((16,), jnp.int32)

    def chunk(c, _):
        base = wid * EPW + c * GB
        pltpu.sync_copy(gi_h.at[pl.ds(base, GB)], ia_v.at[pl.ds(0, GB)])
        pltpu.sync_copy(gj_h.at[pl.ds(base, GB)], ja_v.at[pl.ds(0, GB)])
        cp_a = pltpu.async_copy(xa_h.at[ia_v.at[pl.ds(0, GB)]], ra_v, sem_a)
        cp_b = pltpu.async_copy(xb_h.at[ja_v.at[pl.ds(0, GB)]], rb_v, sem_b)

        def normk(k, _):
            iv = ia_v[pl.ds(k * 16, 16)]
            jv = ja_v[pl.ds(k * 16, 16)]
            dx = plsc.load_gather(px_v, [iv]) - plsc.load_gather(px_v, [jv])
            dy = plsc.load_gather(py_v, [iv]) - plsc.load_gather(py_v, [jv])
            dz = plsc.load_gather(pz_v, [iv]) - plsc.load_gather(pz_v, [jv])
            n2 = dx * dx + dy * dy + dz * dz
            bits = plsc.bitcast(n2, jnp.int32)
            y = plsc.bitcast(jnp.int32(0x5F3759DF) - (bits >> 1), jnp.float32)
            hn2 = 0.5 * n2
            y = y * (1.5 - hn2 * y * y)
            y = y * (1.5 - hn2 * y * y)
            y = y * (1.5 - hn2 * y * y)
            nrm_v[pl.ds(k * 16, 16)] = n2 * y
            return 0

        lax.fori_loop(0, (GB + 15) // 16, normk, 0, unroll=2)
        cp_a.wait()
        cp_b.wait()

        def addrow(r, _):
            nb = plsc.load_gather(nrm_v, [jnp.full((16,), r, jnp.int32)])
            for s in range(D // 16):
                sl = pl.ds(s * 16, 16)
                ra_v[r, sl] = jnp.maximum(
                    ra_v[r, sl] + rb_v[r, sl] + nb * w3r[s], 0.0)
            return 0

        lax.fori_loop(0, GB, addrow, 0, unroll=4)
        pltpu.sync_copy(ra_v, h0_h.at[pl.ds(base, GB), :])
        return 0

    lax.fori_loop(0, EPW // GB, chunk, 0)


def _gather(xa, xb, gi, gj, px, py, pz, w3):
    kfn = functools.partial(
        pl.kernel,
        out_type=jax.ShapeDtypeStruct((E, D), jnp.float32),
        mesh=_mesh,
        scratch_types=[
            pltpu.VMEM((GBP,), jnp.int32),
            pltpu.VMEM((GBP,), jnp.int32),
            pltpu.VMEM((GB, D), jnp.float32),
            pltpu.VMEM((GB, D), jnp.float32),
            pltpu.VMEM((GBP,), jnp.float32),
            pltpu.VMEM((N,), jnp.float32),
            pltpu.VMEM((N,), jnp.float32),
            pltpu.VMEM((N,), jnp.float32),
            pltpu.VMEM((D,), jnp.float32),
            pltpu.SemaphoreType.DMA,
            pltpu.SemaphoreType.DMA,
        ],
        compiler_params=pltpu.CompilerParams(needs_layout_passes=False),
    )(_gather_body)
    return kfn(xa, xb, gi, gj, px, py, pz, w3)


def _edge_mlp_kernel(h0_ref, w1_ref, b1_ref, w2_ref, b2_ref,
                     lng_ref, lnb_ref, out_ref):
    h0 = h0_ref[...]
    h1 = jnp.maximum(
        jnp.dot(h0, w1_ref[...], preferred_element_type=jnp.float32) + b1_ref[...],
        0.0)
    h2 = jnp.dot(h1, w2_ref[...], preferred_element_type=jnp.float32) + b2_ref[...]
    mu = jnp.mean(h2, axis=-1, keepdims=True)
    xc = h2 - mu
    var = jnp.mean(xc * xc, axis=-1, keepdims=True)
    out_ref[...] = xc * lax.rsqrt(var + 1e-5) * lng_ref[...] + lnb_ref[...]


def _edge_mlp(h0, w1, b1, w2, b2, lng, lnb):
    be = 1280
    grid = E // be
    return pl.pallas_call(
        _edge_mlp_kernel,
        grid=(grid,),
        in_specs=[
            pl.BlockSpec((be, D), lambda i: (i, 0)),
            pl.BlockSpec((D, D), lambda i: (0, 0)),
            pl.BlockSpec((1, D), lambda i: (0, 0)),
            pl.BlockSpec((D, D), lambda i: (0, 0)),
            pl.BlockSpec((1, D), lambda i: (0, 0)),
            pl.BlockSpec((1, D), lambda i: (0, 0)),
            pl.BlockSpec((1, D), lambda i: (0, 0)),
        ],
        out_specs=pl.BlockSpec((be, D), lambda i: (i, 0)),
        out_shape=jax.ShapeDtypeStruct((E, D), jnp.float32),
    )(h0, w1, b1, w2, b2, lng, lnb)


def _scatter_body(emb_h, gj_h, acc_h, jb_v, rows_v, z_v, acc_sh, sem):
    c = lax.axis_index("c")
    s = lax.axis_index("s")
    wid = s * NC + c

    def zrow(r, _):
        for k in range(D // 16):
            z_v[r, pl.ds(k * 16, 16)] = jnp.zeros((16,), jnp.float32)
        return 0

    lax.fori_loop(0, 40, zrow, 0)

    def zcp(t, _):
        pltpu.sync_copy(z_v, acc_sh.at[pl.ds(s * RPT + t * 40, 40), :])
        return 0

    lax.fori_loop(0, RPT // 40, zcp, 0)
    plsc.subcore_barrier()

    def chunk(cix, _):
        base = wid * EPW + cix * SB
        pltpu.sync_copy(gj_h.at[pl.ds(base, SB)], jb_v)
        pltpu.sync_copy(emb_h.at[pl.ds(base, SB), :], rows_v)
        pltpu.sync_copy(rows_v, acc_sh.at[jb_v], add=True)
        return 0

    lax.fori_loop(0, EPW // SB, chunk, 0)
    plsc.subcore_barrier()
    pltpu.sync_copy(acc_sh.at[pl.ds(s * RPT, RPT), :],
                    acc_h.at[c, pl.ds(s * RPT, RPT), :])


def _scatter(emb, gj):
    kfn = functools.partial(
        pl.kernel,
        out_type=jax.ShapeDtypeStruct((NC, NPAD, D), jnp.float32),
        mesh=_mesh,
        scratch_types=[
            pltpu.VMEM((SB,), jnp.int32),
            pltpu.VMEM((SB, D), jnp.float32),
            pltpu.VMEM((40, D), jnp.float32),
            pltpu.VMEM_SHARED((NPAD, D), jnp.float32),
            pltpu.SemaphoreType.DMA,
        ],
    )(_scatter_body)
    return kfn(emb, gj)


def _node_mlp_kernel(x_ref, a0_ref, a1_ref, w0a_ref, w0b_ref, b0_ref,
                     w1_ref, b1_ref, w2_ref, b2_ref, lng_ref, lnb_ref,
                     out_ref):
    xv = x_ref[...]
    aggr = a0_ref[...] + a1_ref[...]
    h = jnp.maximum(
        jnp.dot(xv, w0a_ref[...], preferred_element_type=jnp.float32)
        + jnp.dot(aggr, w0b_ref[...], preferred_element_type=jnp.float32)
        + b0_ref[...], 0.0)
    h = jnp.maximum(
        jnp.dot(h, w1_ref[...], preferred_element_type=jnp.float32) + b1_ref[...],
        0.0)
    h = jnp.dot(h, w2_ref[...], preferred_element_type=jnp.float32) + b2_ref[...]
    mu = jnp.mean(h, axis=-1, keepdims=True)
    xc = h - mu
    var = jnp.mean(xc * xc, axis=-1, keepdims=True)
    out_ref[...] = xc * lax.rsqrt(var + 1e-5) * lng_ref[...] + lnb_ref[...] + xv


def _node_mlp(x, a0, a1, w0a, w0b, b0, w1, b1, w2, b2, lng, lnb):
    bn = 2000
    grid = N // bn
    wspec = pl.BlockSpec((D, D), lambda i: (0, 0))
    vspec = pl.BlockSpec((1, D), lambda i: (0, 0))
    nspec = pl.BlockSpec((bn, D), lambda i: (i, 0))
    return pl.pallas_call(
        _node_mlp_kernel,
        grid=(grid,),
        in_specs=[nspec, nspec, nspec, wspec, wspec, vspec, wspec, vspec,
                  wspec, vspec, vspec, vspec],
        out_specs=nspec,
        out_shape=jax.ShapeDtypeStruct((N, D), jnp.float32),
    )(x, a0, a1, w0a, w0b, b0, w1, b1, w2, b2, lng, lnb)


def kernel(x, g, pos,
           edge_w0, edge_b0, edge_w1, edge_b1, edge_w2, edge_b2,
           edge_lng, edge_lnb,
           node_w0, node_b0, node_w1, node_b1, node_w2, node_b2,
           node_lng, node_lnb):
    gi = g[0]
    gj = g[1]
    px = pos[:, 0]
    py = pos[:, 1]
    pz = pos[:, 2]
    pos8 = jnp.pad(pos, ((0, 0), (0, 5)))
    wdir8 = jnp.pad(edge_w0[0:3], ((0, 5), (0, 0)))
    w3 = edge_w0[3:4]
    wxi = edge_w0[4:4 + D]
    wxj = edge_w0[4 + D:4 + 2 * D]

    xa, xb = _pre(x, pos8, wxi, wxj, wdir8, edge_b0.reshape(1, D))
    h0 = _gather(xa, xb, gi, gj, px, py, pz, w3.reshape(D))
    emb = _edge_mlp(h0,
                    edge_w1, edge_b1.reshape(1, D),
                    edge_w2, edge_b2.reshape(1, D),
                    edge_lng.reshape(1, D), edge_lnb.reshape(1, D))
    acc = _scatter(emb, gj)
    out = _node_mlp(x, acc[0, :N], acc[1, :N],
                    node_w0[:D], node_w0[D:], node_b0.reshape(1, D),
                    node_w1, node_b1.reshape(1, D),
                    node_w2, node_b2.reshape(1, D),
                    node_lng.reshape(1, D), node_lnb.reshape(1, D))
    return out


# speedup vs baseline: 1.0022x; 1.0022x over previous
import functools

import jax
import jax.numpy as jnp
from jax import lax
from jax.experimental import pallas as pl
from jax.experimental.pallas import tpu as pltpu
from jax.experimental.pallas import tpu_sc as plsc

N = 10000
E = 320000
D = 128

NC = 2
NS = 16
NW = NC * NS
EPW = E // NW
GB = 200
GBP = 208
SB = 200
NPAD = 10240
RPT = NPAD // NS

_mesh = plsc.VectorSubcoreMesh(core_axis_name="c", subcore_axis_name="s")


def _pre_kernel(x_ref, pos_ref, wxi_ref, wxj_ref, wdir_ref, b0_ref,
                xa_ref, xb_ref):
    pd = jnp.dot(pos_ref[...], wdir_ref[...], preferred_element_type=jnp.float32)
    xv = x_ref[...]
    xa_ref[...] = (jnp.dot(xv, wxi_ref[...], preferred_element_type=jnp.float32)
                   + pd + b0_ref[...])
    xb_ref[...] = (jnp.dot(xv, wxj_ref[...], preferred_element_type=jnp.float32)
                   - pd)


def _pre(x, pos8, wxi, wxj, wdir8, b0):
    bn = 2000
    grid = N // bn
    return pl.pallas_call(
        _pre_kernel,
        grid=(grid,),
        in_specs=[
            pl.BlockSpec((bn, D), lambda i: (i, 0)),
            pl.BlockSpec((bn, 8), lambda i: (i, 0)),
            pl.BlockSpec((D, D), lambda i: (0, 0)),
            pl.BlockSpec((D, D), lambda i: (0, 0)),
            pl.BlockSpec((8, D), lambda i: (0, 0)),
            pl.BlockSpec((1, D), lambda i: (0, 0)),
        ],
        out_specs=[
            pl.BlockSpec((bn, D), lambda i: (i, 0)),
            pl.BlockSpec((bn, D), lambda i: (i, 0)),
        ],
        out_shape=[
            jax.ShapeDtypeStruct((N, D), jnp.float32),
            jax.ShapeDtypeStruct((N, D), jnp.float32),
        ],
    )(x, pos8, wxi, wxj, wdir8, b0)


def _gather_body(xa_h, xb_h, gi_h, gj_h, px_h, py_h, pz_h, w3_h,
                 h0_h,
                 ia_v, ja_v, ra_v, rb_v, nrm_v, px_v, py_v, pz_v, w3_v,
                 sem_a, sem_b):
    wid = lax.axis_index("s") * NC + lax.axis_index("c")
    pltpu.sync_copy(px_h, px_v)
    pltpu.sync_copy(py_h, py_v)
    pltpu.sync_copy(pz_h, pz_v)
    pltpu.sync_copy(w3_h, w3_v)
    w3r = [w3_v[pl.ds(s * 16, 16)] for s in range(D // 16)]
    ia_v[pl.ds(GB - GB % 16, 16)] = jnp.zeros((16,), jnp.int32)
    ja_v[pl.ds(GB - GB % 16, 16)] = jnp.zeros((16,), jnp.int32)

    def chunk(c, _):
        base = wid * EPW + c * GB
        pltpu.sync_copy(gi_h.at[pl.ds(base, GB)], ia_v.at[pl.ds(0, GB)])
        pltpu.sync_copy(gj_h.at[pl.ds(base, GB)], ja_v.at[pl.ds(0, GB)])
        cp_a = pltpu.async_copy(xa_h.at[ia_v.at[pl.ds(0, GB)]], ra_v, sem_a)
        cp_b = pltpu.async_copy(xb_h.at[ja_v.at[pl.ds(0, GB)]], rb_v, sem_b)

        def normk(k, _):
            iv = ia_v[pl.ds(k * 16, 16)]
            jv = ja_v[pl.ds(k * 16, 16)]
            dx = plsc.load_gather(px_v, [iv]) - plsc.load_gather(px_v, [jv])
            dy = plsc.load_gather(py_v, [iv]) - plsc.load_gather(py_v, [jv])
            dz = plsc.load_gather(pz_v, [iv]) - plsc.load_gather(pz_v, [jv])
            n2 = dx * dx + dy * dy + dz * dz
            bits = plsc.bitcast(n2, jnp.int32)
            y = plsc.bitcast(jnp.int32(0x5F3759DF) - (bits >> 1), jnp.float32)
            hn2 = 0.5 * n2
            y = y * (1.5 - hn2 * y * y)
            y = y * (1.5 - hn2 * y * y)
            y = y * (1.5 - hn2 * y * y)
            nrm_v[pl.ds(k * 16, 16)] = n2 * y
            return 0

        lax.fori_loop(0, (GB + 15) // 16, normk, 0, unroll=2)
        cp_a.wait()
        cp_b.wait()

        def addrow(r4, _):
            r0 = r4 * 4
            nbs = [plsc.load_gather(nrm_v, [jnp.full((16,), r0 + u, jnp.int32)])
                   for u in range(4)]
            for u in range(4):
                for s in range(D // 16):
                    sl = pl.ds(s * 16, 16)
                    ra_v[r0 + u, sl] = jnp.maximum(
                        ra_v[r0 + u, sl] + rb_v[r0 + u, sl] + nbs[u] * w3r[s],
                        0.0)
            return 0

        lax.fori_loop(0, GB // 4, addrow, 0)
        pltpu.sync_copy(ra_v, h0_h.at[pl.ds(base, GB), :])
        return 0

    lax.fori_loop(0, EPW // GB, chunk, 0)


def _gather(xa, xb, gi, gj, px, py, pz, w3):
    kfn = functools.partial(
        pl.kernel,
        out_type=jax.ShapeDtypeStruct((E, D), jnp.float32),
        mesh=_mesh,
        scratch_types=[
            pltpu.VMEM((GBP,), jnp.int32),
            pltpu.VMEM((GBP,), jnp.int32),
            pltpu.VMEM((GB, D), jnp.float32),
            pltpu.VMEM((GB, D), jnp.float32),
            pltpu.VMEM((GBP,), jnp.float32),
            pltpu.VMEM((N,), jnp.float32),
            pltpu.VMEM((N,), jnp.float32),
            pltpu.VMEM((N,), jnp.float32),
            pltpu.VMEM((D,), jnp.float32),
            pltpu.SemaphoreType.DMA,
            pltpu.SemaphoreType.DMA,
        ],
        compiler_params=pltpu.CompilerParams(needs_layout_passes=False),
    )(_gather_body)
    return kfn(xa, xb, gi, gj, px, py, pz, w3)


def _edge_mlp_kernel(h0_ref, w1_ref, b1_ref, w2_ref, b2_ref,
                     lng_ref, lnb_ref, out_ref):
    h0 = h0_ref[...]
    h1 = jnp.maximum(
        jnp.dot(h0, w1_ref[...], preferred_element_type=jnp.float32) + b1_ref[...],
        0.0)
    h2 = jnp.dot(h1, w2_ref[...], preferred_element_type=jnp.float32) + b2_ref[...]
    mu = jnp.mean(h2, axis=-1, keepdims=True)
    xc = h2 - mu
    var = jnp.mean(xc * xc, axis=-1, keepdims=True)
    out_ref[...] = xc * lax.rsqrt(var + 1e-5) * lng_ref[...] + lnb_ref[...]


def _edge_mlp(h0, w1, b1, w2, b2, lng, lnb):
    be = 1280
    grid = E // be
    return pl.pallas_call(
        _edge_mlp_kernel,
        grid=(grid,),
        in_specs=[
            pl.BlockSpec((be, D), lambda i: (i, 0)),
            pl.BlockSpec((D, D), lambda i: (0, 0)),
            pl.BlockSpec((1, D), lambda i: (0, 0)),
            pl.BlockSpec((D, D), lambda i: (0, 0)),
            pl.BlockSpec((1, D), lambda i: (0, 0)),
            pl.BlockSpec((1, D), lambda i: (0, 0)),
            pl.BlockSpec((1, D), lambda i: (0, 0)),
        ],
        out_specs=pl.BlockSpec((be, D), lambda i: (i, 0)),
        out_shape=jax.ShapeDtypeStruct((E, D), jnp.float32),
    )(h0, w1, b1, w2, b2, lng, lnb)


def _scatter_body(emb_h, gj_h, acc_h, jb_v, rows_v, z_v, acc_sh, sem):
    c = lax.axis_index("c")
    s = lax.axis_index("s")
    wid = s * NC + c

    def zrow(r, _):
        for k in range(D // 16):
            z_v[r, pl.ds(k * 16, 16)] = jnp.zeros((16,), jnp.float32)
        return 0

    lax.fori_loop(0, 40, zrow, 0)

    def zcp(t, _):
        pltpu.sync_copy(z_v, acc_sh.at[pl.ds(s * RPT + t * 40, 40), :])
        return 0

    lax.fori_loop(0, RPT // 40, zcp, 0)
    plsc.subcore_barrier()

    def chunk(cix, _):
        base = wid * EPW + cix * SB
        pltpu.sync_copy(gj_h.at[pl.ds(base, SB)], jb_v)
        pltpu.sync_copy(emb_h.at[pl.ds(base, SB), :], rows_v)
        pltpu.sync_copy(rows_v, acc_sh.at[jb_v], add=True)
        return 0

    lax.fori_loop(0, EPW // SB, chunk, 0)
    plsc.subcore_barrier()
    pltpu.sync_copy(acc_sh.at[pl.ds(s * RPT, RPT), :],
                    acc_h.at[c, pl.ds(s * RPT, RPT), :])


def _scatter(emb, gj):
    kfn = functools.partial(
        pl.kernel,
        out_type=jax.ShapeDtypeStruct((NC, NPAD, D), jnp.float32),
        mesh=_mesh,
        scratch_types=[
            pltpu.VMEM((SB,), jnp.int32),
            pltpu.VMEM((SB, D), jnp.float32),
            pltpu.VMEM((40, D), jnp.float32),
            pltpu.VMEM_SHARED((NPAD, D), jnp.float32),
            pltpu.SemaphoreType.DMA,
        ],
    )(_scatter_body)
    return kfn(emb, gj)


def _node_mlp_kernel(x_ref, a0_ref, a1_ref, w0a_ref, w0b_ref, b0_ref,
                     w1_ref, b1_ref, w2_ref, b2_ref, lng_ref, lnb_ref,
                     out_ref):
    xv = x_ref[...]
    aggr = a0_ref[...] + a1_ref[...]
    h = jnp.maximum(
        jnp.dot(xv, w0a_ref[...], preferred_element_type=jnp.float32)
        + jnp.dot(aggr, w0b_ref[...], preferred_element_type=jnp.float32)
        + b0_ref[...], 0.0)
    h = jnp.maximum(
        jnp.dot(h, w1_ref[...], preferred_element_type=jnp.float32) + b1_ref[...],
        0.0)
    h = jnp.dot(h, w2_ref[...], preferred_element_type=jnp.float32) + b2_ref[...]
    mu = jnp.mean(h, axis=-1, keepdims=True)
    xc = h - mu
    var = jnp.mean(xc * xc, axis=-1, keepdims=True)
    out_ref[...] = xc * lax.rsqrt(var + 1e-5) * lng_ref[...] + lnb_ref[...] + xv


def _node_mlp(x, a0, a1, w0a, w0b, b0, w1, b1, w2, b2, lng, lnb):
    bn = 2000
    grid = N // bn
    wspec = pl.BlockSpec((D, D), lambda i: (0, 0))
    vspec = pl.BlockSpec((1, D), lambda i: (0, 0))
    nspec = pl.BlockSpec((bn, D), lambda i: (i, 0))
    return pl.pallas_call(
        _node_mlp_kernel,
        grid=(grid,),
        in_specs=[nspec, nspec, nspec, wspec, wspec, vspec, wspec, vspec,
                  wspec, vspec, vspec, vspec],
        out_specs=nspec,
        out_shape=jax.ShapeDtypeStruct((N, D), jnp.float32),
    )(x, a0, a1, w0a, w0b, b0, w1, b1, w2, b2, lng, lnb)


def kernel(x, g, pos,
           edge_w0, edge_b0, edge_w1, edge_b1, edge_w2, edge_b2,
           edge_lng, edge_lnb,
           node_w0, node_b0, node_w1, node_b1, node_w2, node_b2,
           node_lng, node_lnb):
    gi = g[0]
    gj = g[1]
    px = pos[:, 0]
    py = pos[:, 1]
    pz = pos[:, 2]
    pos8 = jnp.pad(pos, ((0, 0), (0, 5)))
    wdir8 = jnp.pad(edge_w0[0:3], ((0, 5), (0, 0)))
    w3 = edge_w0[3:4]
    wxi = edge_w0[4:4 + D]
    wxj = edge_w0[4 + D:4 + 2 * D]

    xa, xb = _pre(x, pos8, wxi, wxj, wdir8, edge_b0.reshape(1, D))
    h0 = _gather(xa, xb, gi, gj, px, py, pz, w3.reshape(D))
    emb = _edge_mlp(h0,
                    edge_w1, edge_b1.reshape(1, D),
                    edge_w2, edge_b2.reshape(1, D),
                    edge_lng.reshape(1, D), edge_lnb.reshape(1, D))
    acc = _scatter(emb, gj)
    out = _node_mlp(x, acc[0, :N], acc[1, :N],
                    node_w0[:D], node_w0[D:], node_b0.reshape(1, D),
                    node_w1, node_b1.reshape(1, D),
                    node_w2, node_b2.reshape(1, D),
                    node_lng.reshape(1, D), node_lnb.reshape(1, D))
    return out


# speedup vs baseline: 1.0024x; 1.0002x over previous
import functools

import jax
import jax.numpy as jnp
from jax import lax
from jax.experimental import pallas as pl
from jax.experimental.pallas import tpu as pltpu
from jax.experimental.pallas import tpu_sc as plsc

N = 10000
E = 320000
D = 128

NC = 2
NS = 16
NW = NC * NS
EPW = E // NW
GB = 200
GBP = 208
SB = 200
NPAD = 10240
RPT = NPAD // NS

_mesh = plsc.VectorSubcoreMesh(core_axis_name="c", subcore_axis_name="s")


def _pre_kernel(x_ref, pos_ref, wxi_ref, wxj_ref, wdir_ref, b0_ref,
                xa_ref, xb_ref):
    pd = jnp.dot(pos_ref[...], wdir_ref[...], preferred_element_type=jnp.float32)
    xv = x_ref[...]
    xa_ref[...] = (jnp.dot(xv, wxi_ref[...], preferred_element_type=jnp.float32)
                   + pd + b0_ref[...])
    xb_ref[...] = (jnp.dot(xv, wxj_ref[...], preferred_element_type=jnp.float32)
                   - pd)


def _pre(x, pos8, wxi, wxj, wdir8, b0):
    bn = 2000
    grid = N // bn
    return pl.pallas_call(
        _pre_kernel,
        grid=(grid,),
        in_specs=[
            pl.BlockSpec((bn, D), lambda i: (i, 0)),
            pl.BlockSpec((bn, 8), lambda i: (i, 0)),
            pl.BlockSpec((D, D), lambda i: (0, 0)),
            pl.BlockSpec((D, D), lambda i: (0, 0)),
            pl.BlockSpec((8, D), lambda i: (0, 0)),
            pl.BlockSpec((1, D), lambda i: (0, 0)),
        ],
        out_specs=[
            pl.BlockSpec((bn, D), lambda i: (i, 0)),
            pl.BlockSpec((bn, D), lambda i: (i, 0)),
        ],
        out_shape=[
            jax.ShapeDtypeStruct((N, D), jnp.float32),
            jax.ShapeDtypeStruct((N, D), jnp.float32),
        ],
    )(x, pos8, wxi, wxj, wdir8, b0)


def _gather_body(xa_h, xb_h, gi_h, gj_h, px_h, py_h, pz_h, w3_h,
                 h0_h,
                 ia_v, ja_v, ra_v, rb_v, nrm_v, px_v, py_v, pz_v, w3_v,
                 sem_a, sem_b):
    wid = lax.axis_index("s") * NC + lax.axis_index("c")
    pltpu.sync_copy(px_h, px_v)
    pltpu.sync_copy(py_h, py_v)
    pltpu.sync_copy(pz_h, pz_v)
    pltpu.sync_copy(w3_h, w3_v)
    w3r = [w3_v[pl.ds(s * 16, 16)] for s in range(D // 16)]

    def chunk(c, _):
        base = wid * EPW + c * GB
        pltpu.sync_copy(gi_h.at[pl.ds(base, GB)], ia_v)
        pltpu.sync_copy(gj_h.at[pl.ds(base, GB)], ja_v)
        cp_a = pltpu.async_copy(xa_h.at[ia_v], ra_v, sem_a)
        cp_b = pltpu.async_copy(xb_h.at[ja_v], rb_v, sem_b)

        def normk(k, _):
            off = jnp.minimum(k * 16, GB - 16)
            iv = ia_v[pl.ds(off, 16)]
            jv = ja_v[pl.ds(off, 16)]
            dx = plsc.load_gather(px_v, [iv]) - plsc.load_gather(px_v, [jv])
            dy = plsc.load_gather(py_v, [iv]) - plsc.load_gather(py_v, [jv])
            dz = plsc.load_gather(pz_v, [iv]) - plsc.load_gather(pz_v, [jv])
            n2 = dx * dx + dy * dy + dz * dz
            bits = plsc.bitcast(n2, jnp.int32)
            y = plsc.bitcast(jnp.int32(0x5F3759DF) - (bits >> 1), jnp.float32)
            hn2 = 0.5 * n2
            y = y * (1.5 - hn2 * y * y)
            y = y * (1.5 - hn2 * y * y)
            y = y * (1.5 - hn2 * y * y)
            nrm_v[pl.ds(off, 16)] = n2 * y
            return 0

        lax.fori_loop(0, (GB + 15) // 16, normk, 0, unroll=2)
        cp_a.wait()
        cp_b.wait()

        def addrow(r4, _):
            r0 = r4 * 4
            nbs = [plsc.load_gather(nrm_v, [jnp.full((16,), r0 + u, jnp.int32)])
                   for u in range(4)]
            for u in range(4):
                for s in range(D // 16):
                    sl = pl.ds(s * 16, 16)
                    ra_v[r0 + u, sl] = jnp.maximum(
                        ra_v[r0 + u, sl] + rb_v[r0 + u, sl] + nbs[u] * w3r[s],
                        0.0)
            return 0

        lax.fori_loop(0, GB // 4, addrow, 0)
        pltpu.sync_copy(ra_v, h0_h.at[pl.ds(base, GB), :])
        return 0

    lax.fori_loop(0, EPW // GB, chunk, 0)


def _gather(xa, xb, gi, gj, px, py, pz, w3):
    kfn = functools.partial(
        pl.kernel,
        out_type=jax.ShapeDtypeStruct((E, D), jnp.float32),
        mesh=_mesh,
        scratch_types=[
            pltpu.VMEM((GB,), jnp.int32),
            pltpu.VMEM((GB,), jnp.int32),
            pltpu.VMEM((GB, D), jnp.float32),
            pltpu.VMEM((GB, D), jnp.float32),
            pltpu.VMEM((GB,), jnp.float32),
            pltpu.VMEM((N,), jnp.float32),
            pltpu.VMEM((N,), jnp.float32),
            pltpu.VMEM((N,), jnp.float32),
            pltpu.VMEM((D,), jnp.float32),
            pltpu.SemaphoreType.DMA,
            pltpu.SemaphoreType.DMA,
        ],
        compiler_params=pltpu.CompilerParams(needs_layout_passes=False),
    )(_gather_body)
    return kfn(xa, xb, gi, gj, px, py, pz, w3)


def _edge_mlp_kernel(h0_ref, w1_ref, b1_ref, w2_ref, b2_ref,
                     lng_ref, lnb_ref, out_ref):
    h0 = h0_ref[...]
    h1 = jnp.maximum(
        jnp.dot(h0, w1_ref[...], preferred_element_type=jnp.float32) + b1_ref[...],
        0.0)
    h2 = jnp.dot(h1, w2_ref[...], preferred_element_type=jnp.float32) + b2_ref[...]
    mu = jnp.mean(h2, axis=-1, keepdims=True)
    xc = h2 - mu
    var = jnp.mean(xc * xc, axis=-1, keepdims=True)
    out_ref[...] = xc * lax.rsqrt(var + 1e-5) * lng_ref[...] + lnb_ref[...]


def _edge_mlp(h0, w1, b1, w2, b2, lng, lnb):
    be = 1280
    grid = E // be
    return pl.pallas_call(
        _edge_mlp_kernel,
        grid=(grid,),
        in_specs=[
            pl.BlockSpec((be, D), lambda i: (i, 0)),
            pl.BlockSpec((D, D), lambda i: (0, 0)),
            pl.BlockSpec((1, D), lambda i: (0, 0)),
            pl.BlockSpec((D, D), lambda i: (0, 0)),
            pl.BlockSpec((1, D), lambda i: (0, 0)),
            pl.BlockSpec((1, D), lambda i: (0, 0)),
            pl.BlockSpec((1, D), lambda i: (0, 0)),
        ],
        out_specs=pl.BlockSpec((be, D), lambda i: (i, 0)),
        out_shape=jax.ShapeDtypeStruct((E, D), jnp.float32),
    )(h0, w1, b1, w2, b2, lng, lnb)


def _scatter_body(emb_h, gj_h, acc_h, jb_v, rows_v, z_v, acc_sh, sem):
    c = lax.axis_index("c")
    s = lax.axis_index("s")
    wid = s * NC + c

    def zrow(r, _):
        for k in range(D // 16):
            z_v[r, pl.ds(k * 16, 16)] = jnp.zeros((16,), jnp.float32)
        return 0

    lax.fori_loop(0, 40, zrow, 0)

    def zcp(t, _):
        pltpu.sync_copy(z_v, acc_sh.at[pl.ds(s * RPT + t * 40, 40), :])
        return 0

    lax.fori_loop(0, RPT // 40, zcp, 0)
    plsc.subcore_barrier()

    def chunk(cix, _):
        base = wid * EPW + cix * SB
        pltpu.sync_copy(gj_h.at[pl.ds(base, SB)], jb_v)
        pltpu.sync_copy(emb_h.at[pl.ds(base, SB), :], rows_v)
        pltpu.sync_copy(rows_v, acc_sh.at[jb_v], add=True)
        return 0

    lax.fori_loop(0, EPW // SB, chunk, 0)
    plsc.subcore_barrier()
    pltpu.sync_copy(acc_sh.at[pl.ds(s * RPT, RPT), :],
                    acc_h.at[c, pl.ds(s * RPT, RPT), :])


def _scatter(emb, gj):
    kfn = functools.partial(
        pl.kernel,
        out_type=jax.ShapeDtypeStruct((NC, NPAD, D), jnp.float32),
        mesh=_mesh,
        scratch_types=[
            pltpu.VMEM((SB,), jnp.int32),
            pltpu.VMEM((SB, D), jnp.float32),
            pltpu.VMEM((40, D), jnp.float32),
            pltpu.VMEM_SHARED((NPAD, D), jnp.float32),
            pltpu.SemaphoreType.DMA,
        ],
    )(_scatter_body)
    return kfn(emb, gj)


def _node_mlp_kernel(x_ref, a0_ref, a1_ref, w0a_ref, w0b_ref, b0_ref,
                     w1_ref, b1_ref, w2_ref, b2_ref, lng_ref, lnb_ref,
                     out_ref):
    xv = x_ref[...]
    aggr = a0_ref[...] + a1_ref[...]
    h = jnp.maximum(
        jnp.dot(xv, w0a_ref[...], preferred_element_type=jnp.float32)
        + jnp.dot(aggr, w0b_ref[...], preferred_element_type=jnp.float32)
        + b0_ref[...], 0.0)
    h = jnp.maximum(
        jnp.dot(h, w1_ref[...], preferred_element_type=jnp.float32) + b1_ref[...],
        0.0)
    h = jnp.dot(h, w2_ref[...], preferred_element_type=jnp.float32) + b2_ref[...]
    mu = jnp.mean(h, axis=-1, keepdims=True)
    xc = h - mu
    var = jnp.mean(xc * xc, axis=-1, keepdims=True)
    out_ref[...] = xc * lax.rsqrt(var + 1e-5) * lng_ref[...] + lnb_ref[...] + xv


def _node_mlp(x, a0, a1, w0a, w0b, b0, w1, b1, w2, b2, lng, lnb):
    bn = 2000
    grid = N // bn
    wspec = pl.BlockSpec((D, D), lambda i: (0, 0))
    vspec = pl.BlockSpec((1, D), lambda i: (0, 0))
    nspec = pl.BlockSpec((bn, D), lambda i: (i, 0))
    return pl.pallas_call(
        _node_mlp_kernel,
        grid=(grid,),
        in_specs=[nspec, nspec, nspec, wspec, wspec, vspec, wspec, vspec,
                  wspec, vspec, vspec, vspec],
        out_specs=nspec,
        out_shape=jax.ShapeDtypeStruct((N, D), jnp.float32),
    )(x, a0, a1, w0a, w0b, b0, w1, b1, w2, b2, lng, lnb)


def kernel(x, g, pos,
           edge_w0, edge_b0, edge_w1, edge_b1, edge_w2, edge_b2,
           edge_lng, edge_lnb,
           node_w0, node_b0, node_w1, node_b1, node_w2, node_b2,
           node_lng, node_lnb):
    gi = g[0]
    gj = g[1]
    px = pos[:, 0]
    py = pos[:, 1]
    pz = pos[:, 2]
    pos8 = jnp.pad(pos, ((0, 0), (0, 5)))
    wdir8 = jnp.pad(edge_w0[0:3], ((0, 5), (0, 0)))
    w3 = edge_w0[3:4]
    wxi = edge_w0[4:4 + D]
    wxj = edge_w0[4 + D:4 + 2 * D]

    xa, xb = _pre(x, pos8, wxi, wxj, wdir8, edge_b0.reshape(1, D))
    h0 = _gather(xa, xb, gi, gj, px, py, pz, w3.reshape(D))
    emb = _edge_mlp(h0,
                    edge_w1, edge_b1.reshape(1, D),
                    edge_w2, edge_b2.reshape(1, D),
                    edge_lng.reshape(1, D), edge_lnb.reshape(1, D))
    acc = _scatter(emb, gj)
    out = _node_mlp(x, acc[0, :N], acc[1, :N],
                    node_w0[:D], node_w0[D:], node_b0.reshape(1, D),
                    node_w1, node_b1.reshape(1, D),
                    node_w2, node_b2.reshape(1, D),
                    node_lng.reshape(1, D), node_lnb.reshape(1, D))
    return out


# speedup vs baseline: 1.4836x; 1.4800x over previous
import functools

import jax
import jax.numpy as jnp
from jax import lax
from jax.experimental import pallas as pl
from jax.experimental.pallas import tpu as pltpu
from jax.experimental.pallas import tpu_sc as plsc

N = 10000
E = 320000
D = 128

NC = 2
NS = 16
NW = NC * NS
EPW = E // NW
GB = 200
GBP = 208
SB = 200
NPAD = 10240
RPT = NPAD // NS

_mesh = plsc.VectorSubcoreMesh(core_axis_name="c", subcore_axis_name="s")


def _pre_kernel(x_ref, pos_ref, wxi_ref, wxj_ref, wdir_ref, b0_ref,
                xa_ref, xb_ref):
    pd = jnp.dot(pos_ref[...], wdir_ref[...], preferred_element_type=jnp.float32)
    xv = x_ref[...]
    xa_ref[...] = (jnp.dot(xv, wxi_ref[...], preferred_element_type=jnp.float32)
                   + pd + b0_ref[...])
    xb_ref[...] = (jnp.dot(xv, wxj_ref[...], preferred_element_type=jnp.float32)
                   - pd)


def _pre(x, pos8, wxi, wxj, wdir8, b0):
    bn = 2000
    grid = N // bn
    return pl.pallas_call(
        _pre_kernel,
        grid=(grid,),
        in_specs=[
            pl.BlockSpec((bn, D), lambda i: (i, 0)),
            pl.BlockSpec((bn, 8), lambda i: (i, 0)),
            pl.BlockSpec((D, D), lambda i: (0, 0)),
            pl.BlockSpec((D, D), lambda i: (0, 0)),
            pl.BlockSpec((8, D), lambda i: (0, 0)),
            pl.BlockSpec((1, D), lambda i: (0, 0)),
        ],
        out_specs=[
            pl.BlockSpec((bn, D), lambda i: (i, 0)),
            pl.BlockSpec((bn, D), lambda i: (i, 0)),
        ],
        out_shape=[
            jax.ShapeDtypeStruct((N, D), jnp.float32),
            jax.ShapeDtypeStruct((N, D), jnp.float32),
        ],
    )(x, pos8, wxi, wxj, wdir8, b0)


def _gather_body(xa_h, xb_h, gi_h, gj_h, px_h, py_h, pz_h, w3_h,
                 h0_h,
                 ia_v, ja_v, ra_v, rb_v, nrm_v, px_v, py_v, pz_v, w3_v,
                 sem_a, sem_b):
    wid = lax.axis_index("s") * NC + lax.axis_index("c")
    pltpu.sync_copy(px_h, px_v)
    pltpu.sync_copy(py_h, py_v)
    pltpu.sync_copy(pz_h, pz_v)
    pltpu.sync_copy(w3_h, w3_v)
    w3r = [w3_v[pl.ds(s * 16, 16)] for s in range(D // 16)]

    def chunk(c, _):
        base = wid * EPW + c * GB
        pltpu.sync_copy(gi_h.at[pl.ds(base, GB)], ia_v)
        pltpu.sync_copy(gj_h.at[pl.ds(base, GB)], ja_v)
        cp_a = pltpu.async_copy(xa_h.at[ia_v], ra_v, sem_a)
        cp_b = pltpu.async_copy(xb_h.at[ja_v], rb_v, sem_b)

        def normk(k, _):
            off = jnp.minimum(k * 16, GB - 16)
            iv = ia_v[pl.ds(off, 16)]
            jv = ja_v[pl.ds(off, 16)]
            dx = plsc.load_gather(px_v, [iv]) - plsc.load_gather(px_v, [jv])
            dy = plsc.load_gather(py_v, [iv]) - plsc.load_gather(py_v, [jv])
            dz = plsc.load_gather(pz_v, [iv]) - plsc.load_gather(pz_v, [jv])
            n2 = dx * dx + dy * dy + dz * dz
            bits = plsc.bitcast(n2, jnp.int32)
            y = plsc.bitcast(jnp.int32(0x5F3759DF) - (bits >> 1), jnp.float32)
            hn2 = 0.5 * n2
            y = y * (1.5 - hn2 * y * y)
            y = y * (1.5 - hn2 * y * y)
            y = y * (1.5 - hn2 * y * y)
            nrm_v[pl.ds(off, 16)] = n2 * y
            return 0

        lax.fori_loop(0, (GB + 15) // 16, normk, 0, unroll=2)
        cp_a.wait()
        cp_b.wait()

        def addrow(r4, _):
            r0 = r4 * 4
            nbs = [plsc.load_gather(nrm_v, [jnp.full((16,), r0 + u, jnp.int32)])
                   for u in range(4)]
            for u in range(4):
                for s in range(D // 16):
                    sl = pl.ds(s * 16, 16)
                    plsc.addupdate(ra_v.at[r0 + u, sl],
                                   rb_v[r0 + u, sl] + nbs[u] * w3r[s])
            return 0

        lax.fori_loop(0, GB // 4, addrow, 0)
        pltpu.sync_copy(ra_v, h0_h.at[pl.ds(base, GB), :])
        return 0

    lax.fori_loop(0, EPW // GB, chunk, 0)


def _gather(xa, xb, gi, gj, px, py, pz, w3):
    kfn = functools.partial(
        pl.kernel,
        out_type=jax.ShapeDtypeStruct((E, D), jnp.float32),
        mesh=_mesh,
        scratch_types=[
            pltpu.VMEM((GB,), jnp.int32),
            pltpu.VMEM((GB,), jnp.int32),
            pltpu.VMEM((GB, D), jnp.float32),
            pltpu.VMEM((GB, D), jnp.float32),
            pltpu.VMEM((GB,), jnp.float32),
            pltpu.VMEM((N,), jnp.float32),
            pltpu.VMEM((N,), jnp.float32),
            pltpu.VMEM((N,), jnp.float32),
            pltpu.VMEM((D,), jnp.float32),
            pltpu.SemaphoreType.DMA,
            pltpu.SemaphoreType.DMA,
        ],
        compiler_params=pltpu.CompilerParams(needs_layout_passes=False),
    )(_gather_body)
    return kfn(xa, xb, gi, gj, px, py, pz, w3)


def _edge_mlp_kernel(h0_ref, w1_ref, b1_ref, w2_ref, b2_ref,
                     lng_ref, lnb_ref, out_ref):
    h0 = jnp.maximum(h0_ref[...], 0.0)
    h1 = jnp.maximum(
        jnp.dot(h0, w1_ref[...], preferred_element_type=jnp.float32) + b1_ref[...],
        0.0)
    h2 = jnp.dot(h1, w2_ref[...], preferred_element_type=jnp.float32) + b2_ref[...]
    mu = jnp.mean(h2, axis=-1, keepdims=True)
    xc = h2 - mu
    var = jnp.mean(xc * xc, axis=-1, keepdims=True)
    out_ref[...] = xc * lax.rsqrt(var + 1e-5) * lng_ref[...] + lnb_ref[...]


def _edge_mlp(h0, w1, b1, w2, b2, lng, lnb):
    be = 1280
    grid = E // be
    return pl.pallas_call(
        _edge_mlp_kernel,
        grid=(grid,),
        in_specs=[
            pl.BlockSpec((be, D), lambda i: (i, 0)),
            pl.BlockSpec((D, D), lambda i: (0, 0)),
            pl.BlockSpec((1, D), lambda i: (0, 0)),
            pl.BlockSpec((D, D), lambda i: (0, 0)),
            pl.BlockSpec((1, D), lambda i: (0, 0)),
            pl.BlockSpec((1, D), lambda i: (0, 0)),
            pl.BlockSpec((1, D), lambda i: (0, 0)),
        ],
        out_specs=pl.BlockSpec((be, D), lambda i: (i, 0)),
        out_shape=jax.ShapeDtypeStruct((E, D), jnp.float32),
    )(h0, w1, b1, w2, b2, lng, lnb)


def _scatter_body(emb_h, gj_h, acc_h, jb_v, rows_v, z_v, acc_sh, sem):
    c = lax.axis_index("c")
    s = lax.axis_index("s")
    wid = s * NC + c

    def zrow(r, _):
        for k in range(D // 16):
            z_v[r, pl.ds(k * 16, 16)] = jnp.zeros((16,), jnp.float32)
        return 0

    lax.fori_loop(0, 40, zrow, 0)

    def zcp(t, _):
        pltpu.sync_copy(z_v, acc_sh.at[pl.ds(s * RPT + t * 40, 40), :])
        return 0

    lax.fori_loop(0, RPT // 40, zcp, 0)
    plsc.subcore_barrier()

    def chunk(cix, _):
        base = wid * EPW + cix * SB
        pltpu.sync_copy(gj_h.at[pl.ds(base, SB)], jb_v)
        pltpu.sync_copy(emb_h.at[pl.ds(base, SB), :], rows_v)
        pltpu.sync_copy(rows_v, acc_sh.at[jb_v], add=True)
        return 0

    lax.fori_loop(0, EPW // SB, chunk, 0)
    plsc.subcore_barrier()
    pltpu.sync_copy(acc_sh.at[pl.ds(s * RPT, RPT), :],
                    acc_h.at[c, pl.ds(s * RPT, RPT), :])


def _scatter(emb, gj):
    kfn = functools.partial(
        pl.kernel,
        out_type=jax.ShapeDtypeStruct((NC, NPAD, D), jnp.float32),
        mesh=_mesh,
        scratch_types=[
            pltpu.VMEM((SB,), jnp.int32),
            pltpu.VMEM((SB, D), jnp.float32),
            pltpu.VMEM((40, D), jnp.float32),
            pltpu.VMEM_SHARED((NPAD, D), jnp.float32),
            pltpu.SemaphoreType.DMA,
        ],
    )(_scatter_body)
    return kfn(emb, gj)


def _node_mlp_kernel(x_ref, a0_ref, a1_ref, w0a_ref, w0b_ref, b0_ref,
                     w1_ref, b1_ref, w2_ref, b2_ref, lng_ref, lnb_ref,
                     out_ref):
    xv = x_ref[...]
    aggr = a0_ref[...] + a1_ref[...]
    h = jnp.maximum(
        jnp.dot(xv, w0a_ref[...], preferred_element_type=jnp.float32)
        + jnp.dot(aggr, w0b_ref[...], preferred_element_type=jnp.float32)
        + b0_ref[...], 0.0)
    h = jnp.maximum(
        jnp.dot(h, w1_ref[...], preferred_element_type=jnp.float32) + b1_ref[...],
        0.0)
    h = jnp.dot(h, w2_ref[...], preferred_element_type=jnp.float32) + b2_ref[...]
    mu = jnp.mean(h, axis=-1, keepdims=True)
    xc = h - mu
    var = jnp.mean(xc * xc, axis=-1, keepdims=True)
    out_ref[...] = xc * lax.rsqrt(var + 1e-5) * lng_ref[...] + lnb_ref[...] + xv


def _node_mlp(x, a0, a1, w0a, w0b, b0, w1, b1, w2, b2, lng, lnb):
    bn = 2000
    grid = N // bn
    wspec = pl.BlockSpec((D, D), lambda i: (0, 0))
    vspec = pl.BlockSpec((1, D), lambda i: (0, 0))
    nspec = pl.BlockSpec((bn, D), lambda i: (i, 0))
    return pl.pallas_call(
        _node_mlp_kernel,
        grid=(grid,),
        in_specs=[nspec, nspec, nspec, wspec, wspec, vspec, wspec, vspec,
                  wspec, vspec, vspec, vspec],
        out_specs=nspec,
        out_shape=jax.ShapeDtypeStruct((N, D), jnp.float32),
    )(x, a0, a1, w0a, w0b, b0, w1, b1, w2, b2, lng, lnb)


def kernel(x, g, pos,
           edge_w0, edge_b0, edge_w1, edge_b1, edge_w2, edge_b2,
           edge_lng, edge_lnb,
           node_w0, node_b0, node_w1, node_b1, node_w2, node_b2,
           node_lng, node_lnb):
    gi = g[0]
    gj = g[1]
    px = pos[:, 0]
    py = pos[:, 1]
    pz = pos[:, 2]
    pos8 = jnp.pad(pos, ((0, 0), (0, 5)))
    wdir8 = jnp.pad(edge_w0[0:3], ((0, 5), (0, 0)))
    w3 = edge_w0[3:4]
    wxi = edge_w0[4:4 + D]
    wxj = edge_w0[4 + D:4 + 2 * D]

    xa, xb = _pre(x, pos8, wxi, wxj, wdir8, edge_b0.reshape(1, D))
    h0 = _gather(xa, xb, gi, gj, px, py, pz, w3.reshape(D))
    emb = _edge_mlp(h0,
                    edge_w1, edge_b1.reshape(1, D),
                    edge_w2, edge_b2.reshape(1, D),
                    edge_lng.reshape(1, D), edge_lnb.reshape(1, D))
    acc = _scatter(emb, gj)
    out = _node_mlp(x, acc[0, :N], acc[1, :N],
                    node_w0[:D], node_w0[D:], node_b0.reshape(1, D),
                    node_w1, node_b1.reshape(1, D),
                    node_w2, node_b2.reshape(1, D),
                    node_lng.reshape(1, D), node_lnb.reshape(1, D))
    return out


# speedup vs baseline: 1.6396x; 1.1051x over previous
import functools

import jax
import jax.numpy as jnp
from jax import lax
from jax.experimental import pallas as pl
from jax.experimental.pallas import tpu as pltpu
from jax.experimental.pallas import tpu_sc as plsc

N = 10000
E = 320000
D = 128

NC = 2
NS = 16
NW = NC * NS
EPW = E // NW
GB = 200
GBP = 208
SB = 200
NPAD = 10240
RPT = NPAD // NS

_mesh = plsc.VectorSubcoreMesh(core_axis_name="c", subcore_axis_name="s")


def _pre_kernel(x_ref, pos_ref, wxi_ref, wxj_ref, wdir_ref, b0_ref,
                xa_ref, xb_ref):
    pd = jnp.dot(pos_ref[...], wdir_ref[...], preferred_element_type=jnp.float32)
    xv = x_ref[...]
    xa_ref[...] = (jnp.dot(xv, wxi_ref[...], preferred_element_type=jnp.float32)
                   + pd + b0_ref[...])
    xb_ref[...] = (jnp.dot(xv, wxj_ref[...], preferred_element_type=jnp.float32)
                   - pd)


def _pre(x, pos8, wxi, wxj, wdir8, b0):
    bn = 2000
    grid = N // bn
    return pl.pallas_call(
        _pre_kernel,
        grid=(grid,),
        in_specs=[
            pl.BlockSpec((bn, D), lambda i: (i, 0)),
            pl.BlockSpec((bn, 8), lambda i: (i, 0)),
            pl.BlockSpec((D, D), lambda i: (0, 0)),
            pl.BlockSpec((D, D), lambda i: (0, 0)),
            pl.BlockSpec((8, D), lambda i: (0, 0)),
            pl.BlockSpec((1, D), lambda i: (0, 0)),
        ],
        out_specs=[
            pl.BlockSpec((bn, D), lambda i: (i, 0)),
            pl.BlockSpec((bn, D), lambda i: (i, 0)),
        ],
        out_shape=[
            jax.ShapeDtypeStruct((N, D), jnp.float32),
            jax.ShapeDtypeStruct((N, D), jnp.float32),
        ],
    )(x, pos8, wxi, wxj, wdir8, b0)


def _gather_body(xa_h, xb_h, gi_h, gj_h, px_h, py_h, pz_h, w3_h,
                 h0_h,
                 ia_v, ja_v, ra_v, rb_v, nrm_v, px_v, py_v, pz_v, w3_v,
                 sem_a, sem_b):
    wid = lax.axis_index("s") * NC + lax.axis_index("c")
    pltpu.sync_copy(px_h, px_v)
    pltpu.sync_copy(py_h, py_v)
    pltpu.sync_copy(pz_h, pz_v)
    pltpu.sync_copy(w3_h, w3_v)
    w3r = [w3_v[pl.ds(s * 16, 16)] for s in range(D // 16)]

    def chunk(c, _):
        base = wid * EPW + c * GB
        pltpu.sync_copy(gi_h.at[pl.ds(base, GB)], ia_v)
        pltpu.sync_copy(gj_h.at[pl.ds(base, GB)], ja_v)
        cp_a = pltpu.async_copy(xa_h.at[ia_v], ra_v, sem_a)
        cp_b = pltpu.async_copy(xb_h.at[ja_v], rb_v, sem_b)

        def normk(k, _):
            off = jnp.minimum(k * 16, GB - 16)
            iv = ia_v[pl.ds(off, 16)]
            jv = ja_v[pl.ds(off, 16)]
            dx = plsc.load_gather(px_v, [iv]) - plsc.load_gather(px_v, [jv])
            dy = plsc.load_gather(py_v, [iv]) - plsc.load_gather(py_v, [jv])
            dz = plsc.load_gather(pz_v, [iv]) - plsc.load_gather(pz_v, [jv])
            n2 = dx * dx + dy * dy + dz * dz
            bits = plsc.bitcast(n2, jnp.int32)
            y = plsc.bitcast(jnp.int32(0x5F3759DF) - (bits >> 1), jnp.float32)
            hn2 = 0.5 * n2
            y = y * (1.5 - hn2 * y * y)
            y = y * (1.5 - hn2 * y * y)
            y = y * (1.5 - hn2 * y * y)
            nrm_v[pl.ds(off, 16)] = n2 * y
            return 0

        lax.fori_loop(0, (GB + 15) // 16, normk, 0, unroll=2)
        cp_a.wait()
        cp_b.wait()

        def addrow(r4, _):
            r0 = r4 * 4
            nbs = [plsc.load_gather(nrm_v, [jnp.full((16,), r0 + u, jnp.int32)])
                   for u in range(4)]
            for u in range(4):
                for s in range(D // 16):
                    sl = pl.ds(s * 16, 16)
                    plsc.addupdate(ra_v.at[r0 + u, sl],
                                   rb_v[r0 + u, sl] + nbs[u] * w3r[s])
            return 0

        lax.fori_loop(0, GB // 4, addrow, 0)
        pltpu.sync_copy(ra_v, h0_h.at[pl.ds(base, GB), :])
        return 0

    lax.fori_loop(0, EPW // GB, chunk, 0)


def _gather(xa, xb, gi, gj, px, py, pz, w3):
    kfn = functools.partial(
        pl.kernel,
        out_type=jax.ShapeDtypeStruct((E, D), jnp.float32),
        mesh=_mesh,
        scratch_types=[
            pltpu.VMEM((GB,), jnp.int32),
            pltpu.VMEM((GB,), jnp.int32),
            pltpu.VMEM((GB, D), jnp.float32),
            pltpu.VMEM((GB, D), jnp.float32),
            pltpu.VMEM((GB,), jnp.float32),
            pltpu.VMEM((N,), jnp.float32),
            pltpu.VMEM((N,), jnp.float32),
            pltpu.VMEM((N,), jnp.float32),
            pltpu.VMEM((D,), jnp.float32),
            pltpu.SemaphoreType.DMA,
            pltpu.SemaphoreType.DMA,
        ],
        compiler_params=pltpu.CompilerParams(needs_layout_passes=False),
    )(_gather_body)
    return kfn(xa, xb, gi, gj, px, py, pz, w3)


def _edge_mlp_kernel(h0_ref, w1_ref, b1_ref, w2_ref, b2_ref,
                     lng_ref, lnb_ref, out_ref):
    h0 = jnp.maximum(h0_ref[...], 0.0)
    h1 = jnp.maximum(
        jnp.dot(h0, w1_ref[...], preferred_element_type=jnp.float32) + b1_ref[...],
        0.0)
    h2 = jnp.dot(h1, w2_ref[...], preferred_element_type=jnp.float32) + b2_ref[...]
    mu = jnp.mean(h2, axis=-1, keepdims=True)
    xc = h2 - mu
    var = jnp.mean(xc * xc, axis=-1, keepdims=True)
    out_ref[...] = xc * lax.rsqrt(var + 1e-5) * lng_ref[...] + lnb_ref[...]


def _edge_mlp(h0, w1, b1, w2, b2, lng, lnb):
    be = 2560
    grid = E // be
    return pl.pallas_call(
        _edge_mlp_kernel,
        grid=(grid,),
        in_specs=[
            pl.BlockSpec((be, D), lambda i: (i, 0)),
            pl.BlockSpec((D, D), lambda i: (0, 0)),
            pl.BlockSpec((1, D), lambda i: (0, 0)),
            pl.BlockSpec((D, D), lambda i: (0, 0)),
            pl.BlockSpec((1, D), lambda i: (0, 0)),
            pl.BlockSpec((1, D), lambda i: (0, 0)),
            pl.BlockSpec((1, D), lambda i: (0, 0)),
        ],
        out_specs=pl.BlockSpec((be, D), lambda i: (i, 0)),
        out_shape=jax.ShapeDtypeStruct((E, D), jnp.float32),
    )(h0, w1, b1, w2, b2, lng, lnb)


def _scatter_body(emb_h, gj_h, acc_h, jb_v, rows_v, z_v, acc_sh, sem):
    c = lax.axis_index("c")
    s = lax.axis_index("s")
    wid = s * NC + c

    def zrow(r, _):
        for k in range(D // 16):
            z_v[r, pl.ds(k * 16, 16)] = jnp.zeros((16,), jnp.float32)
        return 0

    lax.fori_loop(0, 40, zrow, 0)

    def zcp(t, _):
        pltpu.sync_copy(z_v, acc_sh.at[pl.ds(s * RPT + t * 40, 40), :])
        return 0

    lax.fori_loop(0, RPT // 40, zcp, 0)
    plsc.subcore_barrier()

    def chunk(cix, _):
        base = wid * EPW + cix * SB
        pltpu.sync_copy(gj_h.at[pl.ds(base, SB)], jb_v)
        pltpu.sync_copy(emb_h.at[pl.ds(base, SB), :], rows_v)
        pltpu.sync_copy(rows_v, acc_sh.at[jb_v], add=True)
        return 0

    lax.fori_loop(0, EPW // SB, chunk, 0)
    plsc.subcore_barrier()
    pltpu.sync_copy(acc_sh.at[pl.ds(s * RPT, RPT), :],
                    acc_h.at[c, pl.ds(s * RPT, RPT), :])


def _scatter(emb, gj):
    kfn = functools.partial(
        pl.kernel,
        out_type=jax.ShapeDtypeStruct((NC, NPAD, D), jnp.float32),
        mesh=_mesh,
        scratch_types=[
            pltpu.VMEM((SB,), jnp.int32),
            pltpu.VMEM((SB, D), jnp.float32),
            pltpu.VMEM((40, D), jnp.float32),
            pltpu.VMEM_SHARED((NPAD, D), jnp.float32),
            pltpu.SemaphoreType.DMA,
        ],
    )(_scatter_body)
    return kfn(emb, gj)


def _node_mlp_kernel(x_ref, a0_ref, a1_ref, w0a_ref, w0b_ref, b0_ref,
                     w1_ref, b1_ref, w2_ref, b2_ref, lng_ref, lnb_ref,
                     out_ref):
    xv = x_ref[...]
    aggr = a0_ref[...] + a1_ref[...]
    h = jnp.maximum(
        jnp.dot(xv, w0a_ref[...], preferred_element_type=jnp.float32)
        + jnp.dot(aggr, w0b_ref[...], preferred_element_type=jnp.float32)
        + b0_ref[...], 0.0)
    h = jnp.maximum(
        jnp.dot(h, w1_ref[...], preferred_element_type=jnp.float32) + b1_ref[...],
        0.0)
    h = jnp.dot(h, w2_ref[...], preferred_element_type=jnp.float32) + b2_ref[...]
    mu = jnp.mean(h, axis=-1, keepdims=True)
    xc = h - mu
    var = jnp.mean(xc * xc, axis=-1, keepdims=True)
    out_ref[...] = xc * lax.rsqrt(var + 1e-5) * lng_ref[...] + lnb_ref[...] + xv


def _node_mlp(x, a0, a1, w0a, w0b, b0, w1, b1, w2, b2, lng, lnb):
    bn = 2000
    grid = N // bn
    wspec = pl.BlockSpec((D, D), lambda i: (0, 0))
    vspec = pl.BlockSpec((1, D), lambda i: (0, 0))
    nspec = pl.BlockSpec((bn, D), lambda i: (i, 0))
    return pl.pallas_call(
        _node_mlp_kernel,
        grid=(grid,),
        in_specs=[nspec, nspec, nspec, wspec, wspec, vspec, wspec, vspec,
                  wspec, vspec, vspec, vspec],
        out_specs=nspec,
        out_shape=jax.ShapeDtypeStruct((N, D), jnp.float32),
    )(x, a0, a1, w0a, w0b, b0, w1, b1, w2, b2, lng, lnb)


def kernel(x, g, pos,
           edge_w0, edge_b0, edge_w1, edge_b1, edge_w2, edge_b2,
           edge_lng, edge_lnb,
           node_w0, node_b0, node_w1, node_b1, node_w2, node_b2,
           node_lng, node_lnb):
    gi = g[0]
    gj = g[1]
    px = pos[:, 0]
    py = pos[:, 1]
    pz = pos[:, 2]
    pos8 = jnp.pad(pos, ((0, 0), (0, 5)))
    wdir8 = jnp.pad(edge_w0[0:3], ((0, 5), (0, 0)))
    w3 = edge_w0[3:4]
    wxi = edge_w0[4:4 + D]
    wxj = edge_w0[4 + D:4 + 2 * D]

    xa, xb = _pre(x, pos8, wxi, wxj, wdir8, edge_b0.reshape(1, D))
    h0 = _gather(xa, xb, gi, gj, px, py, pz, w3.reshape(D))
    emb = _edge_mlp(h0,
                    edge_w1, edge_b1.reshape(1, D),
                    edge_w2, edge_b2.reshape(1, D),
                    edge_lng.reshape(1, D), edge_lnb.reshape(1, D))
    acc = _scatter(emb, gj)
    out = _node_mlp(x, acc[0, :N], acc[1, :N],
                    node_w0[:D], node_w0[D:], node_b0.reshape(1, D),
                    node_w1, node_b1.reshape(1, D),
                    node_w2, node_b2.reshape(1, D),
                    node_lng.reshape(1, D), node_lnb.reshape(1, D))
    return out


# speedup vs baseline: 1.6756x; 1.0220x over previous
import functools

import jax
import jax.numpy as jnp
from jax import lax
from jax.experimental import pallas as pl
from jax.experimental.pallas import tpu as pltpu
from jax.experimental.pallas import tpu_sc as plsc

N = 10000
E = 320000
D = 128

NC = 2
NS = 16
NW = NC * NS
EPW = E // NW
GB = 200
GBP = 208
SB = 200
NPAD = 10240
RPT = NPAD // NS

_mesh = plsc.VectorSubcoreMesh(core_axis_name="c", subcore_axis_name="s")


def _pre_kernel(x_ref, pos_ref, wxi_ref, wxj_ref, wdir_ref, b0_ref,
                xa_ref, xb_ref):
    pd = jnp.dot(pos_ref[...], wdir_ref[...], preferred_element_type=jnp.float32)
    xv = x_ref[...]
    xa_ref[...] = (jnp.dot(xv, wxi_ref[...], preferred_element_type=jnp.float32)
                   + pd + b0_ref[...])
    xb_ref[...] = (jnp.dot(xv, wxj_ref[...], preferred_element_type=jnp.float32)
                   - pd)


def _pre(x, pos8, wxi, wxj, wdir8, b0):
    bn = 2000
    grid = N // bn
    return pl.pallas_call(
        _pre_kernel,
        grid=(grid,),
        in_specs=[
            pl.BlockSpec((bn, D), lambda i: (i, 0)),
            pl.BlockSpec((bn, 8), lambda i: (i, 0)),
            pl.BlockSpec((D, D), lambda i: (0, 0)),
            pl.BlockSpec((D, D), lambda i: (0, 0)),
            pl.BlockSpec((8, D), lambda i: (0, 0)),
            pl.BlockSpec((1, D), lambda i: (0, 0)),
        ],
        out_specs=[
            pl.BlockSpec((bn, D), lambda i: (i, 0)),
            pl.BlockSpec((bn, D), lambda i: (i, 0)),
        ],
        out_shape=[
            jax.ShapeDtypeStruct((N, D), jnp.float32),
            jax.ShapeDtypeStruct((N, D), jnp.float32),
        ],
    )(x, pos8, wxi, wxj, wdir8, b0)


def _gather_body(xa_h, xb_h, g_h, px_h, py_h, pz_h, w3_h,
                 h0_h,
                 ia_v, ja_v, ra_v, rb_v, nrm_v, px_v, py_v, pz_v, w3_v,
                 sem_a, sem_b):
    wid = lax.axis_index("s") * NC + lax.axis_index("c")
    pltpu.sync_copy(px_h, px_v)
    pltpu.sync_copy(py_h, py_v)
    pltpu.sync_copy(pz_h, pz_v)
    pltpu.sync_copy(w3_h, w3_v)
    w3r = [w3_v[pl.ds(s * 16, 16)] for s in range(D // 16)]

    def chunk(c, _):
        base = wid * EPW + c * GB
        pltpu.sync_copy(g_h.at[pl.ds(base, GB)], ia_v)
        pltpu.sync_copy(g_h.at[pl.ds(E + base, GB)], ja_v)
        cp_a = pltpu.async_copy(xa_h.at[ia_v], ra_v, sem_a)
        cp_b = pltpu.async_copy(xb_h.at[ja_v], rb_v, sem_b)

        def normk(k, _):
            off = jnp.minimum(k * 16, GB - 16)
            iv = ia_v[pl.ds(off, 16)]
            jv = ja_v[pl.ds(off, 16)]
            dx = plsc.load_gather(px_v, [iv]) - plsc.load_gather(px_v, [jv])
            dy = plsc.load_gather(py_v, [iv]) - plsc.load_gather(py_v, [jv])
            dz = plsc.load_gather(pz_v, [iv]) - plsc.load_gather(pz_v, [jv])
            n2 = dx * dx + dy * dy + dz * dz
            bits = plsc.bitcast(n2, jnp.int32)
            y = plsc.bitcast(jnp.int32(0x5F3759DF) - (bits >> 1), jnp.float32)
            hn2 = 0.5 * n2
            y = y * (1.5 - hn2 * y * y)
            y = y * (1.5 - hn2 * y * y)
            y = y * (1.5 - hn2 * y * y)
            nrm_v[pl.ds(off, 16)] = n2 * y
            return 0

        lax.fori_loop(0, (GB + 15) // 16, normk, 0, unroll=2)
        cp_a.wait()
        cp_b.wait()

        def addrow(r4, _):
            r0 = r4 * 4
            nbs = [plsc.load_gather(nrm_v, [jnp.full((16,), r0 + u, jnp.int32)])
                   for u in range(4)]
            for u in range(4):
                for s in range(D // 16):
                    sl = pl.ds(s * 16, 16)
                    plsc.addupdate(ra_v.at[r0 + u, sl],
                                   rb_v[r0 + u, sl] + nbs[u] * w3r[s])
            return 0

        lax.fori_loop(0, GB // 4, addrow, 0)
        pltpu.sync_copy(ra_v, h0_h.at[pl.ds(base, GB), :])
        return 0

    lax.fori_loop(0, EPW // GB, chunk, 0)


def _gather(xa, xb, g, px, py, pz, w3):
    kfn = functools.partial(
        pl.kernel,
        out_type=jax.ShapeDtypeStruct((E, D), jnp.float32),
        mesh=_mesh,
        scratch_types=[
            pltpu.VMEM((GB,), jnp.int32),
            pltpu.VMEM((GB,), jnp.int32),
            pltpu.VMEM((GB, D), jnp.float32),
            pltpu.VMEM((GB, D), jnp.float32),
            pltpu.VMEM((GB,), jnp.float32),
            pltpu.VMEM((N,), jnp.float32),
            pltpu.VMEM((N,), jnp.float32),
            pltpu.VMEM((N,), jnp.float32),
            pltpu.VMEM((D,), jnp.float32),
            pltpu.SemaphoreType.DMA,
            pltpu.SemaphoreType.DMA,
        ],
        compiler_params=pltpu.CompilerParams(needs_layout_passes=False),
    )(_gather_body)
    return kfn(xa, xb, g, px, py, pz, w3)


def _edge_mlp_kernel(h0_ref, w1_ref, b1_ref, w2_ref, b2_ref,
                     lng_ref, lnb_ref, out_ref):
    h0 = jnp.maximum(h0_ref[...], 0.0)
    h1 = jnp.maximum(
        jnp.dot(h0, w1_ref[...], preferred_element_type=jnp.float32) + b1_ref[...],
        0.0)
    h2 = jnp.dot(h1, w2_ref[...], preferred_element_type=jnp.float32) + b2_ref[...]
    mu = jnp.mean(h2, axis=-1, keepdims=True)
    xc = h2 - mu
    var = jnp.mean(xc * xc, axis=-1, keepdims=True)
    out_ref[...] = xc * lax.rsqrt(var + 1e-5) * lng_ref[...] + lnb_ref[...]


def _edge_mlp(h0, w1, b1, w2, b2, lng, lnb):
    be = 2560
    grid = E // be
    return pl.pallas_call(
        _edge_mlp_kernel,
        grid=(grid,),
        in_specs=[
            pl.BlockSpec((be, D), lambda i: (i, 0)),
            pl.BlockSpec((D, D), lambda i: (0, 0)),
            pl.BlockSpec((1, D), lambda i: (0, 0)),
            pl.BlockSpec((D, D), lambda i: (0, 0)),
            pl.BlockSpec((1, D), lambda i: (0, 0)),
            pl.BlockSpec((1, D), lambda i: (0, 0)),
            pl.BlockSpec((1, D), lambda i: (0, 0)),
        ],
        out_specs=pl.BlockSpec((be, D), lambda i: (i, 0)),
        out_shape=jax.ShapeDtypeStruct((E, D), jnp.float32),
    )(h0, w1, b1, w2, b2, lng, lnb)


def _scatter_body(emb_h, g_h, acc_h, jb_v, rows_v, z_v, acc_sh, sem):
    c = lax.axis_index("c")
    s = lax.axis_index("s")
    wid = s * NC + c

    def zrow(r, _):
        for k in range(D // 16):
            z_v[r, pl.ds(k * 16, 16)] = jnp.zeros((16,), jnp.float32)
        return 0

    lax.fori_loop(0, 40, zrow, 0)

    def zcp(t, _):
        pltpu.sync_copy(z_v, acc_sh.at[pl.ds(s * RPT + t * 40, 40), :])
        return 0

    lax.fori_loop(0, RPT // 40, zcp, 0)
    plsc.subcore_barrier()

    def chunk(cix, _):
        base = wid * EPW + cix * SB
        pltpu.sync_copy(g_h.at[pl.ds(E + base, SB)], jb_v)
        pltpu.sync_copy(emb_h.at[pl.ds(base, SB), :], rows_v)
        pltpu.sync_copy(rows_v, acc_sh.at[jb_v], add=True)
        return 0

    lax.fori_loop(0, EPW // SB, chunk, 0)
    plsc.subcore_barrier()
    pltpu.sync_copy(acc_sh.at[pl.ds(s * RPT, RPT), :],
                    acc_h.at[c, pl.ds(s * RPT, RPT), :])


def _scatter(emb, gj):
    kfn = functools.partial(
        pl.kernel,
        out_type=jax.ShapeDtypeStruct((NC, NPAD, D), jnp.float32),
        mesh=_mesh,
        scratch_types=[
            pltpu.VMEM((SB,), jnp.int32),
            pltpu.VMEM((SB, D), jnp.float32),
            pltpu.VMEM((40, D), jnp.float32),
            pltpu.VMEM_SHARED((NPAD, D), jnp.float32),
            pltpu.SemaphoreType.DMA,
        ],
    )(_scatter_body)
    return kfn(emb, gj)


def _node_mlp_kernel(x_ref, a0_ref, a1_ref, w0a_ref, w0b_ref, b0_ref,
                     w1_ref, b1_ref, w2_ref, b2_ref, lng_ref, lnb_ref,
                     out_ref):
    xv = x_ref[...]
    aggr = a0_ref[0] + a1_ref[0]
    h = jnp.maximum(
        jnp.dot(xv, w0a_ref[...], preferred_element_type=jnp.float32)
        + jnp.dot(aggr, w0b_ref[...], preferred_element_type=jnp.float32)
        + b0_ref[...], 0.0)
    h = jnp.maximum(
        jnp.dot(h, w1_ref[...], preferred_element_type=jnp.float32) + b1_ref[...],
        0.0)
    h = jnp.dot(h, w2_ref[...], preferred_element_type=jnp.float32) + b2_ref[...]
    mu = jnp.mean(h, axis=-1, keepdims=True)
    xc = h - mu
    var = jnp.mean(xc * xc, axis=-1, keepdims=True)
    out_ref[...] = xc * lax.rsqrt(var + 1e-5) * lng_ref[...] + lnb_ref[...] + xv


def _node_mlp(x, a0, a1, w0a, w0b, b0, w1, b1, w2, b2, lng, lnb):
    bn = 2000
    grid = N // bn
    wspec = pl.BlockSpec((D, D), lambda i: (0, 0))
    vspec = pl.BlockSpec((1, D), lambda i: (0, 0))
    nspec = pl.BlockSpec((bn, D), lambda i: (i, 0))
    a0spec = pl.BlockSpec((1, bn, D), lambda i: (0, i, 0))
    a1spec = pl.BlockSpec((1, bn, D), lambda i: (1, i, 0))
    return pl.pallas_call(
        _node_mlp_kernel,
        grid=(grid,),
        in_specs=[nspec, a0spec, a1spec, wspec, wspec, vspec, wspec, vspec,
                  wspec, vspec, vspec, vspec],
        out_specs=nspec,
        out_shape=jax.ShapeDtypeStruct((N, D), jnp.float32),
    )(x, a0, a1, w0a, w0b, b0, w1, b1, w2, b2, lng, lnb)


def kernel(x, g, pos,
           edge_w0, edge_b0, edge_w1, edge_b1, edge_w2, edge_b2,
           edge_lng, edge_lnb,
           node_w0, node_b0, node_w1, node_b1, node_w2, node_b2,
           node_lng, node_lnb):
    pos8 = jnp.pad(pos, ((0, 0), (0, 5)))
    wdir8 = jnp.pad(edge_w0[0:3], ((0, 5), (0, 0)))
    w3 = edge_w0[3:4]
    wxi = edge_w0[4:4 + D]
    wxj = edge_w0[4 + D:4 + 2 * D]

    xa, xb = _pre(x, pos8, wxi, wxj, wdir8, edge_b0.reshape(1, D))
    gflat = g.reshape(2 * E)
    h0 = _gather(xa, xb, gflat, pos[:, 0], pos[:, 1], pos[:, 2],
                 w3.reshape(D))
    emb = _edge_mlp(h0,
                    edge_w1, edge_b1.reshape(1, D),
                    edge_w2, edge_b2.reshape(1, D),
                    edge_lng.reshape(1, D), edge_lnb.reshape(1, D))
    acc = _scatter(emb, gflat)
    out = _node_mlp(x, acc, acc,
                    node_w0[:D], node_w0[D:], node_b0.reshape(1, D),
                    node_w1, node_b1.reshape(1, D),
                    node_w2, node_b2.reshape(1, D),
                    node_lng.reshape(1, D), node_lnb.reshape(1, D))
    return out


# speedup vs baseline: 1.7543x; 1.0469x over previous
import functools

import jax
import jax.numpy as jnp
from jax import lax
from jax.experimental import pallas as pl
from jax.experimental.pallas import tpu as pltpu
from jax.experimental.pallas import tpu_sc as plsc

N = 10000
E = 320000
D = 128

NC = 2
NS = 16
NW = NC * NS
EPW = E // NW
GB = 200
GBP = 208
SB = 200
NPAD = 10240
RPT = NPAD // NS

_mesh = plsc.VectorSubcoreMesh(core_axis_name="c", subcore_axis_name="s")


def _pre_kernel(x_ref, pos_ref, wxi_ref, wxj_ref, wdir_ref, b0_ref,
                xa_ref, xb_ref):
    pd = jnp.dot(pos_ref[...], wdir_ref[...], preferred_element_type=jnp.float32)
    xv = x_ref[...]
    xa_ref[...] = (jnp.dot(xv, wxi_ref[...], preferred_element_type=jnp.float32)
                   + pd + b0_ref[...])
    xb_ref[...] = (jnp.dot(xv, wxj_ref[...], preferred_element_type=jnp.float32)
                   - pd)


def _pre(x, pos8, wxi, wxj, wdir8, b0):
    bn = 2000
    grid = N // bn
    return pl.pallas_call(
        _pre_kernel,
        grid=(grid,),
        in_specs=[
            pl.BlockSpec((bn, D), lambda i: (i, 0)),
            pl.BlockSpec((bn, 8), lambda i: (i, 0)),
            pl.BlockSpec((D, D), lambda i: (0, 0)),
            pl.BlockSpec((D, D), lambda i: (0, 0)),
            pl.BlockSpec((8, D), lambda i: (0, 0)),
            pl.BlockSpec((1, D), lambda i: (0, 0)),
        ],
        out_specs=[
            pl.BlockSpec((bn, D), lambda i: (i, 0)),
            pl.BlockSpec((bn, D), lambda i: (i, 0)),
        ],
        out_shape=[
            jax.ShapeDtypeStruct((N, D), jnp.float32),
            jax.ShapeDtypeStruct((N, D), jnp.float32),
        ],
    )(x, pos8, wxi, wxj, wdir8, b0)


def _gather_body(xa_h, xb_h, g_h, px_h, py_h, pz_h, w3_h,
                 h0_h,
                 ia_v, ja_v, ra_v, rb_v, nrm_v, px_v, py_v, pz_v, w3_v,
                 sem_a, sem_b):
    wid = lax.axis_index("s") * NC + lax.axis_index("c")
    pltpu.sync_copy(px_h, px_v)
    pltpu.sync_copy(py_h, py_v)
    pltpu.sync_copy(pz_h, pz_v)
    pltpu.sync_copy(w3_h, w3_v)
    w3r = [w3_v[pl.ds(s * 16, 16)] for s in range(D // 16)]

    def chunk(c, _):
        base = wid * EPW + c * GB
        pltpu.sync_copy(g_h.at[pl.ds(base, GB)], ia_v)
        pltpu.sync_copy(g_h.at[pl.ds(E + base, GB)], ja_v)
        cp_a = pltpu.async_copy(xa_h.at[ia_v], ra_v, sem_a)
        cp_b = pltpu.async_copy(xb_h.at[ja_v], rb_v, sem_b)

        def normk(k, _):
            off = jnp.minimum(k * 16, GB - 16)
            iv = ia_v[pl.ds(off, 16)]
            jv = ja_v[pl.ds(off, 16)]
            dx = plsc.load_gather(px_v, [iv]) - plsc.load_gather(px_v, [jv])
            dy = plsc.load_gather(py_v, [iv]) - plsc.load_gather(py_v, [jv])
            dz = plsc.load_gather(pz_v, [iv]) - plsc.load_gather(pz_v, [jv])
            n2 = dx * dx + dy * dy + dz * dz
            bits = plsc.bitcast(n2, jnp.int32)
            y = plsc.bitcast(jnp.int32(0x5F3759DF) - (bits >> 1), jnp.float32)
            hn2 = 0.5 * n2
            y = y * (1.5 - hn2 * y * y)
            y = y * (1.5 - hn2 * y * y)
            y = y * (1.5 - hn2 * y * y)
            nrm_v[pl.ds(off, 16)] = n2 * y
            return 0

        lax.fori_loop(0, (GB + 15) // 16, normk, 0, unroll=2)
        cp_a.wait()
        cp_b.wait()

        def addrow(r4, _):
            r0 = r4 * 4
            nbs = [plsc.load_gather(nrm_v, [jnp.full((16,), r0 + u, jnp.int32)])
                   for u in range(4)]
            for u in range(4):
                for s in range(D // 16):
                    sl = pl.ds(s * 16, 16)
                    plsc.addupdate(ra_v.at[r0 + u, sl],
                                   rb_v[r0 + u, sl] + nbs[u] * w3r[s])
            return 0

        lax.fori_loop(0, GB // 4, addrow, 0)
        pltpu.sync_copy(ra_v, h0_h.at[pl.ds(base, GB), :])
        return 0

    lax.fori_loop(0, EPW // GB, chunk, 0)


def _gather(xa, xb, g, px, py, pz, w3):
    kfn = functools.partial(
        pl.kernel,
        out_type=jax.ShapeDtypeStruct((E, D), jnp.float32),
        mesh=_mesh,
        scratch_types=[
            pltpu.VMEM((GB,), jnp.int32),
            pltpu.VMEM((GB,), jnp.int32),
            pltpu.VMEM((GB, D), jnp.float32),
            pltpu.VMEM((GB, D), jnp.float32),
            pltpu.VMEM((GB,), jnp.float32),
            pltpu.VMEM((N,), jnp.float32),
            pltpu.VMEM((N,), jnp.float32),
            pltpu.VMEM((N,), jnp.float32),
            pltpu.VMEM((D,), jnp.float32),
            pltpu.SemaphoreType.DMA,
            pltpu.SemaphoreType.DMA,
        ],
        compiler_params=pltpu.CompilerParams(needs_layout_passes=False),
    )(_gather_body)
    return kfn(xa, xb, g, px, py, pz, w3)


def _edge_mlp_kernel(h0_ref, w1_ref, b1_ref, w2_ref, b2_ref,
                     lng_ref, lnb_ref, out_ref):
    h0 = jnp.maximum(h0_ref[...], 0.0)
    h1 = jnp.maximum(
        jnp.dot(h0, w1_ref[...], preferred_element_type=jnp.float32) + b1_ref[...],
        0.0)
    h2 = jnp.dot(h1, w2_ref[...], preferred_element_type=jnp.float32) + b2_ref[...]
    mu = jnp.mean(h2, axis=-1, keepdims=True)
    xc = h2 - mu
    var = jnp.mean(xc * xc, axis=-1, keepdims=True)
    out_ref[...] = xc * lax.rsqrt(var + 1e-5) * lng_ref[...] + lnb_ref[...]


def _edge_mlp(h0, w1, b1, w2, b2, lng, lnb):
    be = 4000
    grid = E // be
    return pl.pallas_call(
        _edge_mlp_kernel,
        grid=(grid,),
        in_specs=[
            pl.BlockSpec((be, D), lambda i: (i, 0)),
            pl.BlockSpec((D, D), lambda i: (0, 0)),
            pl.BlockSpec((1, D), lambda i: (0, 0)),
            pl.BlockSpec((D, D), lambda i: (0, 0)),
            pl.BlockSpec((1, D), lambda i: (0, 0)),
            pl.BlockSpec((1, D), lambda i: (0, 0)),
            pl.BlockSpec((1, D), lambda i: (0, 0)),
        ],
        out_specs=pl.BlockSpec((be, D), lambda i: (i, 0)),
        out_shape=jax.ShapeDtypeStruct((E, D), jnp.float32),
    )(h0, w1, b1, w2, b2, lng, lnb)


def _scatter_body(emb_h, g_h, acc_h, jb_v, rows_v, z_v, acc_sh, sem):
    c = lax.axis_index("c")
    s = lax.axis_index("s")
    wid = s * NC + c

    def zrow(r, _):
        for k in range(D // 16):
            z_v[r, pl.ds(k * 16, 16)] = jnp.zeros((16,), jnp.float32)
        return 0

    lax.fori_loop(0, 40, zrow, 0)

    def zcp(t, _):
        pltpu.sync_copy(z_v, acc_sh.at[pl.ds(s * RPT + t * 40, 40), :])
        return 0

    lax.fori_loop(0, RPT // 40, zcp, 0)
    plsc.subcore_barrier()

    def chunk(cix, _):
        base = wid * EPW + cix * SB
        pltpu.sync_copy(g_h.at[pl.ds(E + base, SB)], jb_v)
        pltpu.sync_copy(emb_h.at[pl.ds(base, SB), :], rows_v)
        pltpu.sync_copy(rows_v, acc_sh.at[jb_v], add=True)
        return 0

    lax.fori_loop(0, EPW // SB, chunk, 0)
    plsc.subcore_barrier()
    pltpu.sync_copy(acc_sh.at[pl.ds(s * RPT, RPT), :],
                    acc_h.at[c, pl.ds(s * RPT, RPT), :])


def _scatter(emb, gj):
    kfn = functools.partial(
        pl.kernel,
        out_type=jax.ShapeDtypeStruct((NC, NPAD, D), jnp.float32),
        mesh=_mesh,
        scratch_types=[
            pltpu.VMEM((SB,), jnp.int32),
            pltpu.VMEM((SB, D), jnp.float32),
            pltpu.VMEM((40, D), jnp.float32),
            pltpu.VMEM_SHARED((NPAD, D), jnp.float32),
            pltpu.SemaphoreType.DMA,
        ],
    )(_scatter_body)
    return kfn(emb, gj)


def _node_mlp_kernel(x_ref, a0_ref, a1_ref, w0a_ref, w0b_ref, b0_ref,
                     w1_ref, b1_ref, w2_ref, b2_ref, lng_ref, lnb_ref,
                     out_ref):
    xv = x_ref[...]
    aggr = a0_ref[0] + a1_ref[0]
    h = jnp.maximum(
        jnp.dot(xv, w0a_ref[...], preferred_element_type=jnp.float32)
        + jnp.dot(aggr, w0b_ref[...], preferred_element_type=jnp.float32)
        + b0_ref[...], 0.0)
    h = jnp.maximum(
        jnp.dot(h, w1_ref[...], preferred_element_type=jnp.float32) + b1_ref[...],
        0.0)
    h = jnp.dot(h, w2_ref[...], preferred_element_type=jnp.float32) + b2_ref[...]
    mu = jnp.mean(h, axis=-1, keepdims=True)
    xc = h - mu
    var = jnp.mean(xc * xc, axis=-1, keepdims=True)
    out_ref[...] = xc * lax.rsqrt(var + 1e-5) * lng_ref[...] + lnb_ref[...] + xv


def _node_mlp(x, a0, a1, w0a, w0b, b0, w1, b1, w2, b2, lng, lnb):
    bn = 2000
    grid = N // bn
    wspec = pl.BlockSpec((D, D), lambda i: (0, 0))
    vspec = pl.BlockSpec((1, D), lambda i: (0, 0))
    nspec = pl.BlockSpec((bn, D), lambda i: (i, 0))
    a0spec = pl.BlockSpec((1, bn, D), lambda i: (0, i, 0))
    a1spec = pl.BlockSpec((1, bn, D), lambda i: (1, i, 0))
    return pl.pallas_call(
        _node_mlp_kernel,
        grid=(grid,),
        in_specs=[nspec, a0spec, a1spec, wspec, wspec, vspec, wspec, vspec,
                  wspec, vspec, vspec, vspec],
        out_specs=nspec,
        out_shape=jax.ShapeDtypeStruct((N, D), jnp.float32),
    )(x, a0, a1, w0a, w0b, b0, w1, b1, w2, b2, lng, lnb)


def kernel(x, g, pos,
           edge_w0, edge_b0, edge_w1, edge_b1, edge_w2, edge_b2,
           edge_lng, edge_lnb,
           node_w0, node_b0, node_w1, node_b1, node_w2, node_b2,
           node_lng, node_lnb):
    pos8 = jnp.pad(pos, ((0, 0), (0, 5)))
    wdir8 = jnp.pad(edge_w0[0:3], ((0, 5), (0, 0)))
    w3 = edge_w0[3:4]
    wxi = edge_w0[4:4 + D]
    wxj = edge_w0[4 + D:4 + 2 * D]

    xa, xb = _pre(x, pos8, wxi, wxj, wdir8, edge_b0.reshape(1, D))
    gflat = g.reshape(2 * E)
    h0 = _gather(xa, xb, gflat, pos[:, 0], pos[:, 1], pos[:, 2],
                 w3.reshape(D))
    emb = _edge_mlp(h0,
                    edge_w1, edge_b1.reshape(1, D),
                    edge_w2, edge_b2.reshape(1, D),
                    edge_lng.reshape(1, D), edge_lnb.reshape(1, D))
    acc = _scatter(emb, gflat)
    out = _node_mlp(x, acc, acc,
                    node_w0[:D], node_w0[D:], node_b0.reshape(1, D),
                    node_w1, node_b1.reshape(1, D),
                    node_w2, node_b2.reshape(1, D),
                    node_lng.reshape(1, D), node_lnb.reshape(1, D))
    return out


# speedup vs baseline: 1.8160x; 1.0352x over previous
import functools

import jax
import jax.numpy as jnp
from jax import lax
from jax.experimental import pallas as pl
from jax.experimental.pallas import tpu as pltpu
from jax.experimental.pallas import tpu_sc as plsc

N = 10000
E = 320000
D = 128

NC = 2
NS = 16
NW = NC * NS
EPW = E // NW
GB = 200
GBP = 208
SB = 200
NPAD = 10240
RPT = NPAD // NS

_mesh = plsc.VectorSubcoreMesh(core_axis_name="c", subcore_axis_name="s")


def _pre_kernel(x_ref, pos_ref, wxi_ref, wxj_ref, wdir_ref, b0_ref,
                xa_ref, xb_ref):
    pd = jnp.dot(pos_ref[...], wdir_ref[...], preferred_element_type=jnp.float32)
    xv = x_ref[...]
    xa_ref[...] = (jnp.dot(xv, wxi_ref[...], preferred_element_type=jnp.float32)
                   + pd + b0_ref[...])
    xb_ref[...] = (jnp.dot(xv, wxj_ref[...], preferred_element_type=jnp.float32)
                   - pd)


def _pre(x, pos8, wxi, wxj, wdir8, b0):
    bn = 2000
    grid = N // bn
    return pl.pallas_call(
        _pre_kernel,
        grid=(grid,),
        in_specs=[
            pl.BlockSpec((bn, D), lambda i: (i, 0)),
            pl.BlockSpec((bn, 8), lambda i: (i, 0)),
            pl.BlockSpec((D, D), lambda i: (0, 0)),
            pl.BlockSpec((D, D), lambda i: (0, 0)),
            pl.BlockSpec((8, D), lambda i: (0, 0)),
            pl.BlockSpec((1, D), lambda i: (0, 0)),
        ],
        out_specs=[
            pl.BlockSpec((bn, D), lambda i: (i, 0)),
            pl.BlockSpec((bn, D), lambda i: (i, 0)),
        ],
        out_shape=[
            jax.ShapeDtypeStruct((N, D), jnp.float32),
            jax.ShapeDtypeStruct((N, D), jnp.float32),
        ],
    )(x, pos8, wxi, wxj, wdir8, b0)


def _gather_body(xa_h, xb_h, g_h, px_h, py_h, pz_h, w3_h,
                 h0_h,
                 ia_v, ja_v, ra_v, rb_v, nrm_v, px_v, py_v, pz_v, w3_v,
                 sem_a, sem_b):
    wid = lax.axis_index("s") * NC + lax.axis_index("c")
    pltpu.sync_copy(px_h, px_v)
    pltpu.sync_copy(py_h, py_v)
    pltpu.sync_copy(pz_h, pz_v)
    pltpu.sync_copy(w3_h, w3_v)
    w3r = [w3_v[pl.ds(s * 16, 16)] for s in range(D // 16)]

    def chunk(c, _):
        base = wid * EPW + c * GB
        pltpu.sync_copy(g_h.at[pl.ds(base, GB)], ia_v)
        pltpu.sync_copy(g_h.at[pl.ds(E + base, GB)], ja_v)
        cp_a = pltpu.async_copy(xa_h.at[ia_v], ra_v, sem_a)
        cp_b = pltpu.async_copy(xb_h.at[ja_v], rb_v, sem_b)

        def normk(k, _):
            off = jnp.minimum(k * 16, GB - 16)
            iv = ia_v[pl.ds(off, 16)]
            jv = ja_v[pl.ds(off, 16)]
            dx = plsc.load_gather(px_v, [iv]) - plsc.load_gather(px_v, [jv])
            dy = plsc.load_gather(py_v, [iv]) - plsc.load_gather(py_v, [jv])
            dz = plsc.load_gather(pz_v, [iv]) - plsc.load_gather(pz_v, [jv])
            n2 = dx * dx + dy * dy + dz * dz
            bits = plsc.bitcast(n2, jnp.int32)
            y = plsc.bitcast(jnp.int32(0x5F3759DF) - (bits >> 1), jnp.float32)
            hn2 = 0.5 * n2
            y = y * (1.5 - hn2 * y * y)
            y = y * (1.5 - hn2 * y * y)
            y = y * (1.5 - hn2 * y * y)
            nrm_v[pl.ds(off, 16)] = n2 * y
            return 0

        lax.fori_loop(0, (GB + 15) // 16, normk, 0, unroll=2)
        cp_a.wait()
        cp_b.wait()

        def addrow(r4, _):
            r0 = r4 * 4
            nbs = [plsc.load_gather(nrm_v, [jnp.full((16,), r0 + u, jnp.int32)])
                   for u in range(4)]
            for u in range(4):
                for s in range(D // 16):
                    sl = pl.ds(s * 16, 16)
                    plsc.addupdate(ra_v.at[r0 + u, sl],
                                   rb_v[r0 + u, sl] + nbs[u] * w3r[s])
            return 0

        lax.fori_loop(0, GB // 4, addrow, 0)
        pltpu.sync_copy(ra_v, h0_h.at[pl.ds(base, GB), :])
        return 0

    lax.fori_loop(0, EPW // GB, chunk, 0)


def _gather(xa, xb, g, px, py, pz, w3):
    kfn = functools.partial(
        pl.kernel,
        out_type=jax.ShapeDtypeStruct((E, D), jnp.float32),
        mesh=_mesh,
        scratch_types=[
            pltpu.VMEM((GB,), jnp.int32),
            pltpu.VMEM((GB,), jnp.int32),
            pltpu.VMEM((GB, D), jnp.float32),
            pltpu.VMEM((GB, D), jnp.float32),
            pltpu.VMEM((GB,), jnp.float32),
            pltpu.VMEM((N,), jnp.float32),
            pltpu.VMEM((N,), jnp.float32),
            pltpu.VMEM((N,), jnp.float32),
            pltpu.VMEM((D,), jnp.float32),
            pltpu.SemaphoreType.DMA,
            pltpu.SemaphoreType.DMA,
        ],
        compiler_params=pltpu.CompilerParams(needs_layout_passes=False),
    )(_gather_body)
    return kfn(xa, xb, g, px, py, pz, w3)


def _edge_mlp_kernel(h0_ref, w1_ref, b1_ref, w2_ref, b2_ref,
                     lng_ref, lnb_ref, out_ref):
    h0 = jnp.maximum(h0_ref[...], 0.0)
    h1 = jnp.maximum(
        jnp.dot(h0, w1_ref[...], preferred_element_type=jnp.float32) + b1_ref[...],
        0.0)
    h2 = jnp.dot(h1, w2_ref[...], preferred_element_type=jnp.float32) + b2_ref[...]
    mu = jnp.mean(h2, axis=-1, keepdims=True)
    xc = h2 - mu
    var = jnp.mean(xc * xc, axis=-1, keepdims=True)
    out_ref[...] = xc * lax.rsqrt(var + 1e-5) * lng_ref[...] + lnb_ref[...]


def _edge_mlp(h0, w1, b1, w2, b2, lng, lnb):
    be = 8000
    grid = E // be
    return pl.pallas_call(
        _edge_mlp_kernel,
        grid=(grid,),
        in_specs=[
            pl.BlockSpec((be, D), lambda i: (i, 0)),
            pl.BlockSpec((D, D), lambda i: (0, 0)),
            pl.BlockSpec((1, D), lambda i: (0, 0)),
            pl.BlockSpec((D, D), lambda i: (0, 0)),
            pl.BlockSpec((1, D), lambda i: (0, 0)),
            pl.BlockSpec((1, D), lambda i: (0, 0)),
            pl.BlockSpec((1, D), lambda i: (0, 0)),
        ],
        out_specs=pl.BlockSpec((be, D), lambda i: (i, 0)),
        out_shape=jax.ShapeDtypeStruct((E, D), jnp.float32),
    )(h0, w1, b1, w2, b2, lng, lnb)


def _scatter_body(emb_h, g_h, acc_h, jb_v, rows_v, z_v, acc_sh, sem):
    c = lax.axis_index("c")
    s = lax.axis_index("s")
    wid = s * NC + c

    def zrow(r, _):
        for k in range(D // 16):
            z_v[r, pl.ds(k * 16, 16)] = jnp.zeros((16,), jnp.float32)
        return 0

    lax.fori_loop(0, 40, zrow, 0)

    def zcp(t, _):
        pltpu.sync_copy(z_v, acc_sh.at[pl.ds(s * RPT + t * 40, 40), :])
        return 0

    lax.fori_loop(0, RPT // 40, zcp, 0)
    plsc.subcore_barrier()

    def chunk(cix, _):
        base = wid * EPW + cix * SB
        pltpu.sync_copy(g_h.at[pl.ds(E + base, SB)], jb_v)
        pltpu.sync_copy(emb_h.at[pl.ds(base, SB), :], rows_v)
        pltpu.sync_copy(rows_v, acc_sh.at[jb_v], add=True)
        return 0

    lax.fori_loop(0, EPW // SB, chunk, 0)
    plsc.subcore_barrier()
    pltpu.sync_copy(acc_sh.at[pl.ds(s * RPT, RPT), :],
                    acc_h.at[c, pl.ds(s * RPT, RPT), :])


def _scatter(emb, gj):
    kfn = functools.partial(
        pl.kernel,
        out_type=jax.ShapeDtypeStruct((NC, NPAD, D), jnp.float32),
        mesh=_mesh,
        scratch_types=[
            pltpu.VMEM((SB,), jnp.int32),
            pltpu.VMEM((SB, D), jnp.float32),
            pltpu.VMEM((40, D), jnp.float32),
            pltpu.VMEM_SHARED((NPAD, D), jnp.float32),
            pltpu.SemaphoreType.DMA,
        ],
    )(_scatter_body)
    return kfn(emb, gj)


def _node_mlp_kernel(x_ref, a0_ref, a1_ref, w0a_ref, w0b_ref, b0_ref,
                     w1_ref, b1_ref, w2_ref, b2_ref, lng_ref, lnb_ref,
                     out_ref):
    xv = x_ref[...]
    aggr = a0_ref[0] + a1_ref[0]
    h = jnp.maximum(
        jnp.dot(xv, w0a_ref[...], preferred_element_type=jnp.float32)
        + jnp.dot(aggr, w0b_ref[...], preferred_element_type=jnp.float32)
        + b0_ref[...], 0.0)
    h = jnp.maximum(
        jnp.dot(h, w1_ref[...], preferred_element_type=jnp.float32) + b1_ref[...],
        0.0)
    h = jnp.dot(h, w2_ref[...], preferred_element_type=jnp.float32) + b2_ref[...]
    mu = jnp.mean(h, axis=-1, keepdims=True)
    xc = h - mu
    var = jnp.mean(xc * xc, axis=-1, keepdims=True)
    out_ref[...] = xc * lax.rsqrt(var + 1e-5) * lng_ref[...] + lnb_ref[...] + xv


def _node_mlp(x, a0, a1, w0a, w0b, b0, w1, b1, w2, b2, lng, lnb):
    bn = 2000
    grid = N // bn
    wspec = pl.BlockSpec((D, D), lambda i: (0, 0))
    vspec = pl.BlockSpec((1, D), lambda i: (0, 0))
    nspec = pl.BlockSpec((bn, D), lambda i: (i, 0))
    a0spec = pl.BlockSpec((1, bn, D), lambda i: (0, i, 0))
    a1spec = pl.BlockSpec((1, bn, D), lambda i: (1, i, 0))
    return pl.pallas_call(
        _node_mlp_kernel,
        grid=(grid,),
        in_specs=[nspec, a0spec, a1spec, wspec, wspec, vspec, wspec, vspec,
                  wspec, vspec, vspec, vspec],
        out_specs=nspec,
        out_shape=jax.ShapeDtypeStruct((N, D), jnp.float32),
    )(x, a0, a1, w0a, w0b, b0, w1, b1, w2, b2, lng, lnb)


def kernel(x, g, pos,
           edge_w0, edge_b0, edge_w1, edge_b1, edge_w2, edge_b2,
           edge_lng, edge_lnb,
           node_w0, node_b0, node_w1, node_b1, node_w2, node_b2,
           node_lng, node_lnb):
    pos8 = jnp.pad(pos, ((0, 0), (0, 5)))
    wdir8 = jnp.pad(edge_w0[0:3], ((0, 5), (0, 0)))
    w3 = edge_w0[3:4]
    wxi = edge_w0[4:4 + D]
    wxj = edge_w0[4 + D:4 + 2 * D]

    xa, xb = _pre(x, pos8, wxi, wxj, wdir8, edge_b0.reshape(1, D))
    gflat = g.reshape(2 * E)
    h0 = _gather(xa, xb, gflat, pos[:, 0], pos[:, 1], pos[:, 2],
                 w3.reshape(D))
    emb = _edge_mlp(h0,
                    edge_w1, edge_b1.reshape(1, D),
                    edge_w2, edge_b2.reshape(1, D),
                    edge_lng.reshape(1, D), edge_lnb.reshape(1, D))
    acc = _scatter(emb, gflat)
    out = _node_mlp(x, acc, acc,
                    node_w0[:D], node_w0[D:], node_b0.reshape(1, D),
                    node_w1, node_b1.reshape(1, D),
                    node_w2, node_b2.reshape(1, D),
                    node_lng.reshape(1, D), node_lnb.reshape(1, D))
    return out


# speedup vs baseline: 1.8510x; 1.0193x over previous
import functools

import jax
import jax.numpy as jnp
from jax import lax
from jax.experimental import pallas as pl
from jax.experimental.pallas import tpu as pltpu
from jax.experimental.pallas import tpu_sc as plsc

N = 10000
E = 320000
D = 128

NC = 2
NS = 16
NW = NC * NS
EPW = E // NW
GB = 200
GBP = 208
SB = 200
NPAD = 10240
RPT = NPAD // NS

_mesh = plsc.VectorSubcoreMesh(core_axis_name="c", subcore_axis_name="s")


def _pre_kernel(x_ref, pos_ref, wxi_ref, wxj_ref, wdir_ref, b0_ref,
                xa_ref, xb_ref):
    pd = jnp.dot(pos_ref[...], wdir_ref[...], preferred_element_type=jnp.float32)
    xv = x_ref[...]
    xa_ref[...] = (jnp.dot(xv, wxi_ref[...], preferred_element_type=jnp.float32)
                   + pd + b0_ref[...])
    xb_ref[...] = (jnp.dot(xv, wxj_ref[...], preferred_element_type=jnp.float32)
                   - pd)


def _pre(x, pos8, wxi, wxj, wdir8, b0):
    bn = 2000
    grid = N // bn
    return pl.pallas_call(
        _pre_kernel,
        grid=(grid,),
        in_specs=[
            pl.BlockSpec((bn, D), lambda i: (i, 0)),
            pl.BlockSpec((bn, 8), lambda i: (i, 0)),
            pl.BlockSpec((D, D), lambda i: (0, 0)),
            pl.BlockSpec((D, D), lambda i: (0, 0)),
            pl.BlockSpec((8, D), lambda i: (0, 0)),
            pl.BlockSpec((1, D), lambda i: (0, 0)),
        ],
        out_specs=[
            pl.BlockSpec((bn, D), lambda i: (i, 0)),
            pl.BlockSpec((bn, D), lambda i: (i, 0)),
        ],
        out_shape=[
            jax.ShapeDtypeStruct((N, D), jnp.float32),
            jax.ShapeDtypeStruct((N, D), jnp.float32),
        ],
    )(x, pos8, wxi, wxj, wdir8, b0)


def _gather_body(xa_h, xb_h, g_h, px_h, py_h, pz_h, w3_h,
                 h0_h,
                 ia_v, ja_v, ra_v, rb_v, nrm_v, px_v, py_v, pz_v, w3_v,
                 sem_a, sem_b):
    wid = lax.axis_index("s") * NC + lax.axis_index("c")
    pltpu.sync_copy(px_h, px_v)
    pltpu.sync_copy(py_h, py_v)
    pltpu.sync_copy(pz_h, pz_v)
    pltpu.sync_copy(w3_h, w3_v)
    w3r = [w3_v[pl.ds(s * 16, 16)] for s in range(D // 16)]

    def chunk(c, _):
        base = wid * EPW + c * GB
        pltpu.sync_copy(g_h.at[pl.ds(base, GB)], ia_v)
        pltpu.sync_copy(g_h.at[pl.ds(E + base, GB)], ja_v)
        cp_a = pltpu.async_copy(xa_h.at[ia_v], ra_v, sem_a)
        cp_b = pltpu.async_copy(xb_h.at[ja_v], rb_v, sem_b)

        def normk(k, _):
            off = jnp.minimum(k * 16, GB - 16)
            iv = ia_v[pl.ds(off, 16)]
            jv = ja_v[pl.ds(off, 16)]
            dx = plsc.load_gather(px_v, [iv]) - plsc.load_gather(px_v, [jv])
            dy = plsc.load_gather(py_v, [iv]) - plsc.load_gather(py_v, [jv])
            dz = plsc.load_gather(pz_v, [iv]) - plsc.load_gather(pz_v, [jv])
            n2 = dx * dx + dy * dy + dz * dz
            bits = plsc.bitcast(n2, jnp.int32)
            y = plsc.bitcast(jnp.int32(0x5F3759DF) - (bits >> 1), jnp.float32)
            hn2 = 0.5 * n2
            y = y * (1.5 - hn2 * y * y)
            y = y * (1.5 - hn2 * y * y)
            y = y * (1.5 - hn2 * y * y)
            nrm_v[pl.ds(off, 16)] = n2 * y
            return 0

        lax.fori_loop(0, (GB + 15) // 16, normk, 0, unroll=2)
        cp_a.wait()
        cp_b.wait()

        def addrow(r4, _):
            r0 = r4 * 4
            nbs = [plsc.load_gather(nrm_v, [jnp.full((16,), r0 + u, jnp.int32)])
                   for u in range(4)]
            for u in range(4):
                for s in range(D // 16):
                    sl = pl.ds(s * 16, 16)
                    plsc.addupdate(ra_v.at[r0 + u, sl],
                                   rb_v[r0 + u, sl] + nbs[u] * w3r[s])
            return 0

        lax.fori_loop(0, GB // 4, addrow, 0)
        pltpu.sync_copy(ra_v, h0_h.at[pl.ds(base, GB), :])
        return 0

    lax.fori_loop(0, EPW // GB, chunk, 0)


def _gather(xa, xb, g, px, py, pz, w3):
    kfn = functools.partial(
        pl.kernel,
        out_type=jax.ShapeDtypeStruct((E, D), jnp.float32),
        mesh=_mesh,
        scratch_types=[
            pltpu.VMEM((GB,), jnp.int32),
            pltpu.VMEM((GB,), jnp.int32),
            pltpu.VMEM((GB, D), jnp.float32),
            pltpu.VMEM((GB, D), jnp.float32),
            pltpu.VMEM((GB,), jnp.float32),
            pltpu.VMEM((N,), jnp.float32),
            pltpu.VMEM((N,), jnp.float32),
            pltpu.VMEM((N,), jnp.float32),
            pltpu.VMEM((D,), jnp.float32),
            pltpu.SemaphoreType.DMA,
            pltpu.SemaphoreType.DMA,
        ],
        compiler_params=pltpu.CompilerParams(needs_layout_passes=False),
    )(_gather_body)
    return kfn(xa, xb, g, px, py, pz, w3)


def _edge_mlp_kernel(h0_ref, w1_ref, b1_ref, w2_ref, b2_ref,
                     lng_ref, lnb_ref, out_ref):
    h0 = jnp.maximum(h0_ref[...], 0.0)
    h1 = jnp.maximum(
        jnp.dot(h0, w1_ref[...], preferred_element_type=jnp.float32) + b1_ref[...],
        0.0)
    h2 = jnp.dot(h1, w2_ref[...], preferred_element_type=jnp.float32) + b2_ref[...]
    mu = jnp.mean(h2, axis=-1, keepdims=True)
    xc = h2 - mu
    var = jnp.mean(xc * xc, axis=-1, keepdims=True)
    out_ref[...] = xc * lax.rsqrt(var + 1e-5) * lng_ref[...] + lnb_ref[...]


def _edge_mlp(h0, w1, b1, w2, b2, lng, lnb):
    be = 16000
    grid = E // be
    return pl.pallas_call(
        _edge_mlp_kernel,
        grid=(grid,),
        in_specs=[
            pl.BlockSpec((be, D), lambda i: (i, 0)),
            pl.BlockSpec((D, D), lambda i: (0, 0)),
            pl.BlockSpec((1, D), lambda i: (0, 0)),
            pl.BlockSpec((D, D), lambda i: (0, 0)),
            pl.BlockSpec((1, D), lambda i: (0, 0)),
            pl.BlockSpec((1, D), lambda i: (0, 0)),
            pl.BlockSpec((1, D), lambda i: (0, 0)),
        ],
        out_specs=pl.BlockSpec((be, D), lambda i: (i, 0)),
        out_shape=jax.ShapeDtypeStruct((E, D), jnp.float32),
    )(h0, w1, b1, w2, b2, lng, lnb)


def _scatter_body(emb_h, g_h, acc_h, jb_v, rows_v, z_v, acc_sh, sem):
    c = lax.axis_index("c")
    s = lax.axis_index("s")
    wid = s * NC + c

    def zrow(r, _):
        for k in range(D // 16):
            z_v[r, pl.ds(k * 16, 16)] = jnp.zeros((16,), jnp.float32)
        return 0

    lax.fori_loop(0, 40, zrow, 0)

    def zcp(t, _):
        pltpu.sync_copy(z_v, acc_sh.at[pl.ds(s * RPT + t * 40, 40), :])
        return 0

    lax.fori_loop(0, RPT // 40, zcp, 0)
    plsc.subcore_barrier()

    def chunk(cix, _):
        base = wid * EPW + cix * SB
        pltpu.sync_copy(g_h.at[pl.ds(E + base, SB)], jb_v)
        pltpu.sync_copy(emb_h.at[pl.ds(base, SB), :], rows_v)
        pltpu.sync_copy(rows_v, acc_sh.at[jb_v], add=True)
        return 0

    lax.fori_loop(0, EPW // SB, chunk, 0)
    plsc.subcore_barrier()
    pltpu.sync_copy(acc_sh.at[pl.ds(s * RPT, RPT), :],
                    acc_h.at[c, pl.ds(s * RPT, RPT), :])


def _scatter(emb, gj):
    kfn = functools.partial(
        pl.kernel,
        out_type=jax.ShapeDtypeStruct((NC, NPAD, D), jnp.float32),
        mesh=_mesh,
        scratch_types=[
            pltpu.VMEM((SB,), jnp.int32),
            pltpu.VMEM((SB, D), jnp.float32),
            pltpu.VMEM((40, D), jnp.float32),
            pltpu.VMEM_SHARED((NPAD, D), jnp.float32),
            pltpu.SemaphoreType.DMA,
        ],
    )(_scatter_body)
    return kfn(emb, gj)


def _node_mlp_kernel(x_ref, a0_ref, a1_ref, w0a_ref, w0b_ref, b0_ref,
                     w1_ref, b1_ref, w2_ref, b2_ref, lng_ref, lnb_ref,
                     out_ref):
    xv = x_ref[...]
    aggr = a0_ref[0] + a1_ref[0]
    h = jnp.maximum(
        jnp.dot(xv, w0a_ref[...], preferred_element_type=jnp.float32)
        + jnp.dot(aggr, w0b_ref[...], preferred_element_type=jnp.float32)
        + b0_ref[...], 0.0)
    h = jnp.maximum(
        jnp.dot(h, w1_ref[...], preferred_element_type=jnp.float32) + b1_ref[...],
        0.0)
    h = jnp.dot(h, w2_ref[...], preferred_element_type=jnp.float32) + b2_ref[...]
    mu = jnp.mean(h, axis=-1, keepdims=True)
    xc = h - mu
    var = jnp.mean(xc * xc, axis=-1, keepdims=True)
    out_ref[...] = xc * lax.rsqrt(var + 1e-5) * lng_ref[...] + lnb_ref[...] + xv


def _node_mlp(x, a0, a1, w0a, w0b, b0, w1, b1, w2, b2, lng, lnb):
    bn = 2000
    grid = N // bn
    wspec = pl.BlockSpec((D, D), lambda i: (0, 0))
    vspec = pl.BlockSpec((1, D), lambda i: (0, 0))
    nspec = pl.BlockSpec((bn, D), lambda i: (i, 0))
    a0spec = pl.BlockSpec((1, bn, D), lambda i: (0, i, 0))
    a1spec = pl.BlockSpec((1, bn, D), lambda i: (1, i, 0))
    return pl.pallas_call(
        _node_mlp_kernel,
        grid=(grid,),
        in_specs=[nspec, a0spec, a1spec, wspec, wspec, vspec, wspec, vspec,
                  wspec, vspec, vspec, vspec],
        out_specs=nspec,
        out_shape=jax.ShapeDtypeStruct((N, D), jnp.float32),
    )(x, a0, a1, w0a, w0b, b0, w1, b1, w2, b2, lng, lnb)


def kernel(x, g, pos,
           edge_w0, edge_b0, edge_w1, edge_b1, edge_w2, edge_b2,
           edge_lng, edge_lnb,
           node_w0, node_b0, node_w1, node_b1, node_w2, node_b2,
           node_lng, node_lnb):
    pos8 = jnp.pad(pos, ((0, 0), (0, 5)))
    wdir8 = jnp.pad(edge_w0[0:3], ((0, 5), (0, 0)))
    w3 = edge_w0[3:4]
    wxi = edge_w0[4:4 + D]
    wxj = edge_w0[4 + D:4 + 2 * D]

    xa, xb = _pre(x, pos8, wxi, wxj, wdir8, edge_b0.reshape(1, D))
    gflat = g.reshape(2 * E)
    h0 = _gather(xa, xb, gflat, pos[:, 0], pos[:, 1], pos[:, 2],
                 w3.reshape(D))
    emb = _edge_mlp(h0,
                    edge_w1, edge_b1.reshape(1, D),
                    edge_w2, edge_b2.reshape(1, D),
                    edge_lng.reshape(1, D), edge_lnb.reshape(1, D))
    acc = _scatter(emb, gflat)
    out = _node_mlp(x, acc, acc,
                    node_w0[:D], node_w0[D:], node_b0.reshape(1, D),
                    node_w1, node_b1.reshape(1, D),
                    node_w2, node_b2.reshape(1, D),
                    node_lng.reshape(1, D), node_lnb.reshape(1, D))
    return out
